# Initial kernel scaffold; baseline (speedup 1.0000x reference)
import jax
import jax.numpy as jnp
from jax.experimental import pallas as pl


def kernel(x, edge_index, attr, W1, b1, W2, b2, W3, b3, Wg, bg):
    raise NotImplementedError("write your pallas kernel here")


# speedup vs baseline: 23.7981x; 23.7981x over previous
import functools

import jax
import jax.numpy as jnp
from jax import lax
from jax.experimental import pallas as pl
from jax.experimental.pallas import tpu as pltpu
from jax.experimental.pallas import tpu_sc as plsc

_LRELU_SLOPE = 0.01


def _lrelu(x):
    return jnp.where(x >= 0, x, _LRELU_SLOPE * x)


def _zero_vmem_1d(zb_v, n):
    def body(i, _):
        zb_v[pl.ds(i * 16, 16)] = jnp.zeros((16,), jnp.float32)
        return 0
    lax.fori_loop(0, n // 16, body, 0)


def _zero_vmem_rows(zb_v, n):
    def body(i, _):
        zb_v[i, :] = jnp.zeros((16,), jnp.float32)
        return 0
    lax.fori_loop(0, n, body, 0)


def _make_sc_degree(rows, npad, chb):
    rpw = rows // 32
    zb = npad // 128
    stripe = npad // 16
    mesh = plsc.VectorSubcoreMesh(core_axis_name="c", subcore_axis_name="s")

    @functools.partial(
        pl.kernel,
        mesh=mesh,
        compiler_params=pltpu.CompilerParams(use_tc_tiling_on_sc=False),
        out_type=[jax.ShapeDtypeStruct((npad,), jnp.float32)] * 2,
        scratch_types=[
            pltpu.VMEM_SHARED((npad,), jnp.float32),
            pltpu.VMEM((zb,), jnp.float32),
            pltpu.VMEM((chb, 128), jnp.int32),
            pltpu.VMEM((chb, 128), jnp.float32),
        ],
    )
    def k(src_hbm, attr_hbm, d0_hbm, d1_hbm, deg_sp, zb_v, src_v, attr_v):
        c = lax.axis_index("c")
        s = lax.axis_index("s")
        wid = c * 16 + s

        _zero_vmem_1d(zb_v, zb)
        off = s * stripe
        for r in range(8):
            pltpu.sync_copy(zb_v, deg_sp.at[pl.ds(off + r * zb, zb)])
        plsc.subcore_barrier()

        base = wid * rpw

        def mloop(m, _):
            r0 = base + m * chb
            pltpu.sync_copy(src_hbm.at[pl.ds(r0, chb)], src_v)
            pltpu.sync_copy(attr_hbm.at[pl.ds(r0, chb)], attr_v)
            for j in range(chb):
                pltpu.sync_copy(attr_v.at[j], deg_sp.at[src_v.at[j]],
                                add=True)
            return 0

        lax.fori_loop(0, rpw // chb, mloop, 0)
        plsc.subcore_barrier()

        @pl.when(c == 0)
        def _():
            pltpu.sync_copy(deg_sp.at[pl.ds(off, stripe)],
                            d0_hbm.at[pl.ds(off, stripe)])

        @pl.when(c == 1)
        def _():
            pltpu.sync_copy(deg_sp.at[pl.ds(off, stripe)],
                            d1_hbm.at[pl.ds(off, stripe)])

    return k


def _make_sc_q1(rows, npad, chb):
    rpw = rows // 16
    zb = npad // 128
    stripe = npad // 16
    mesh = plsc.VectorSubcoreMesh(core_axis_name="c", subcore_axis_name="s")

    @functools.partial(
        pl.kernel,
        mesh=mesh,
        compiler_params=pltpu.CompilerParams(use_tc_tiling_on_sc=False),
        out_type=[jax.ShapeDtypeStruct((npad, 16), jnp.float32)] * 2,
        scratch_types=[
            pltpu.VMEM_SHARED((npad, 16), jnp.float32),
            pltpu.VMEM((zb, 16), jnp.float32),
            pltpu.VMEM((chb, 128), jnp.int32),
            pltpu.VMEM((chb, 128), jnp.int32),
            pltpu.VMEM((chb, 128), jnp.float32),
            pltpu.VMEM((128, 16), jnp.float32),
        ],
    )
    def k(src_hbm, dst_hbm, attr_hbm, g1f_hbm, qa_hbm, qb_hbm,
          q_sp, zb_v, src_v, dst_v, attr_v, rows_v):
        c = lax.axis_index("c")
        s = lax.axis_index("s")

        _zero_vmem_rows(zb_v, zb)
        off = s * stripe
        for r in range(8):
            pltpu.sync_copy(zb_v, q_sp.at[pl.ds(off + r * zb, zb)])
        plsc.subcore_barrier()

        base = s * rpw
        coff = jnp.full((16,), c * npad, dtype=jnp.int32)

        def mloop(m, _):
            r0 = base + m * chb
            pltpu.sync_copy(src_hbm.at[pl.ds(r0, chb)], src_v)
            pltpu.sync_copy(dst_hbm.at[pl.ds(r0, chb)], dst_v)
            pltpu.sync_copy(attr_hbm.at[pl.ds(r0, chb)], attr_v)
            for j in range(chb):
                for g in range(8):
                    sl = pl.ds(g * 16, 16)
                    src_v[j, sl] = src_v[j, sl] + coff
            for j in range(chb):
                pltpu.sync_copy(g1f_hbm.at[src_v.at[j]], rows_v)
                for g in range(8):
                    av = attr_v[j, pl.ds(g * 16, 16)]
                    for li in range(16):
                        i = g * 16 + li
                        rows_v[i, :] = rows_v[i, :] * av[li]
                pltpu.sync_copy(rows_v, q_sp.at[dst_v.at[j]], add=True)
            return 0

        lax.fori_loop(0, rpw // chb, mloop, 0)
        plsc.subcore_barrier()

        @pl.when(c == 0)
        def _():
            pltpu.sync_copy(q_sp.at[pl.ds(off, stripe)],
                            qa_hbm.at[pl.ds(off, stripe)])

        @pl.when(c == 1)
        def _():
            pltpu.sync_copy(q_sp.at[pl.ds(off, stripe)],
                            qb_hbm.at[pl.ds(off, stripe)])

    return k


def _make_sc_q2(rows, npad, chb):
    rpw = rows // 32
    zb = npad // 128
    stripe = npad // 16
    mesh = plsc.VectorSubcoreMesh(core_axis_name="c", subcore_axis_name="s")

    @functools.partial(
        pl.kernel,
        mesh=mesh,
        compiler_params=pltpu.CompilerParams(use_tc_tiling_on_sc=False),
        out_type=[jax.ShapeDtypeStruct((npad,), jnp.float32)] * 4,
        scratch_types=[
            pltpu.VMEM_SHARED((npad,), jnp.float32),
            pltpu.VMEM_SHARED((npad,), jnp.float32),
            pltpu.VMEM_SHARED((npad,), jnp.float32),
            pltpu.VMEM_SHARED((npad,), jnp.float32),
            pltpu.VMEM((zb,), jnp.float32),
            pltpu.VMEM((chb, 128), jnp.int32),
            pltpu.VMEM((chb, 128), jnp.int32),
            pltpu.VMEM((chb, 128), jnp.float32),
            pltpu.VMEM((128,), jnp.float32),
            pltpu.VMEM((128,), jnp.float32),
        ],
    )
    def k(src_hbm, dst_hbm, attr_hbm, g20_hbm, g21_hbm,
          o00_hbm, o01_hbm, o10_hbm, o11_hbm,
          g0_sp, g1_sp, q0_sp, q1_sp, zb_v, src_v, dst_v, attr_v,
          r0_v, r1_v):
        c = lax.axis_index("c")
        s = lax.axis_index("s")
        wid = c * 16 + s

        off = s * stripe
        pltpu.sync_copy(g20_hbm.at[pl.ds(off, stripe)],
                        g0_sp.at[pl.ds(off, stripe)])
        pltpu.sync_copy(g21_hbm.at[pl.ds(off, stripe)],
                        g1_sp.at[pl.ds(off, stripe)])
        _zero_vmem_1d(zb_v, zb)
        for r in range(8):
            pltpu.sync_copy(zb_v, q0_sp.at[pl.ds(off + r * zb, zb)])
            pltpu.sync_copy(zb_v, q1_sp.at[pl.ds(off + r * zb, zb)])
        plsc.subcore_barrier()

        base = wid * rpw

        def mloop(m, _):
            r0 = base + m * chb
            pltpu.sync_copy(src_hbm.at[pl.ds(r0, chb)], src_v)
            pltpu.sync_copy(dst_hbm.at[pl.ds(r0, chb)], dst_v)
            pltpu.sync_copy(attr_hbm.at[pl.ds(r0, chb)], attr_v)
            for j in range(chb):
                pltpu.sync_copy(g0_sp.at[src_v.at[j]], r0_v)
                pltpu.sync_copy(g1_sp.at[src_v.at[j]], r1_v)
                for g in range(8):
                    sl = pl.ds(g * 16, 16)
                    a = attr_v[j, sl]
                    r0_v[sl] = r0_v[sl] * a
                    r1_v[sl] = r1_v[sl] * a
                pltpu.sync_copy(r0_v, q0_sp.at[dst_v.at[j]], add=True)
                pltpu.sync_copy(r1_v, q1_sp.at[dst_v.at[j]], add=True)
            return 0

        lax.fori_loop(0, rpw // chb, mloop, 0)
        plsc.subcore_barrier()

        @pl.when(c == 0)
        def _():
            pltpu.sync_copy(q0_sp.at[pl.ds(off, stripe)],
                            o00_hbm.at[pl.ds(off, stripe)])
            pltpu.sync_copy(q1_sp.at[pl.ds(off, stripe)],
                            o10_hbm.at[pl.ds(off, stripe)])

        @pl.when(c == 1)
        def _():
            pltpu.sync_copy(q0_sp.at[pl.ds(off, stripe)],
                            o01_hbm.at[pl.ds(off, stripe)])
            pltpu.sync_copy(q1_sp.at[pl.ds(off, stripe)],
                            o11_hbm.at[pl.ds(off, stripe)])

    return k


_BLK = 2048


def _tc1_body(x_ref, d0_ref, d1_ref, w_ref, b_ref,
              h1_ref, ga_ref, gb_ref, dis_ref):
    deg = d0_ref[...] + d1_ref[...]
    dis = jnp.where(deg > 0, lax.rsqrt(deg), 0.0)
    h = jnp.dot(x_ref[...], w_ref[...],
                preferred_element_type=jnp.float32) + b_ref[...]
    h1 = _lrelu(h)
    h1_ref[...] = h1
    g = dis * h1
    ga_ref[...] = g[:, :16]
    gb_ref[...] = g[:, 16:]
    dis_ref[...] = dis


def _tc1(xp, d0, d1, W10, b1, npad):
    grid = npad // _BLK
    return pl.pallas_call(
        _tc1_body,
        grid=(grid,),
        in_specs=[
            pl.BlockSpec((_BLK, 20), lambda i: (i, 0)),
            pl.BlockSpec((_BLK, 1), lambda i: (i, 0)),
            pl.BlockSpec((_BLK, 1), lambda i: (i, 0)),
            pl.BlockSpec((20, 32), lambda i: (0, 0)),
            pl.BlockSpec((1, 32), lambda i: (0, 0)),
        ],
        out_specs=[
            pl.BlockSpec((_BLK, 32), lambda i: (i, 0)),
            pl.BlockSpec((_BLK, 16), lambda i: (i, 0)),
            pl.BlockSpec((_BLK, 16), lambda i: (i, 0)),
            pl.BlockSpec((_BLK, 1), lambda i: (i, 0)),
        ],
        out_shape=[
            jax.ShapeDtypeStruct((npad, 32), jnp.float32),
            jax.ShapeDtypeStruct((npad, 16), jnp.float32),
            jax.ShapeDtypeStruct((npad, 16), jnp.float32),
            jax.ShapeDtypeStruct((npad, 1), jnp.float32),
        ],
    )(xp, d0, d1, W10, b1)


def _tc2_body(h1_ref, qa_ref, qb_ref, dis_ref,
              w20_ref, w21_ref, b2_ref, w30_ref, w31_ref, b3_ref,
              h3a_ref, g20_ref, g21_ref):
    dis = dis_ref[...]
    q1 = jnp.concatenate([qa_ref[...], qb_ref[...]], axis=1)
    p1 = -dis * q1
    h2 = _lrelu(
        jnp.dot(h1_ref[...], w20_ref[...],
                preferred_element_type=jnp.float32)
        + jnp.dot(p1, w21_ref[...], preferred_element_type=jnp.float32)
        + b2_ref[...])
    t = jnp.dot(h2, w31_ref[...], preferred_element_type=jnp.float32)
    g2 = dis * t
    g20_ref[...] = g2[:, 0:1]
    g21_ref[...] = g2[:, 1:2]
    h3a_ref[...] = jnp.dot(h2, w30_ref[...],
                           preferred_element_type=jnp.float32) + b3_ref[...]


def _tc2(h1, qa, qb, dis, W20, W21, b2, W30, W31, b3, npad):
    grid = npad // _BLK
    return pl.pallas_call(
        _tc2_body,
        grid=(grid,),
        in_specs=[
            pl.BlockSpec((_BLK, 32), lambda i: (i, 0)),
            pl.BlockSpec((_BLK, 16), lambda i: (i, 0)),
            pl.BlockSpec((_BLK, 16), lambda i: (i, 0)),
            pl.BlockSpec((_BLK, 1), lambda i: (i, 0)),
            pl.BlockSpec((32, 64), lambda i: (0, 0)),
            pl.BlockSpec((32, 64), lambda i: (0, 0)),
            pl.BlockSpec((1, 64), lambda i: (0, 0)),
            pl.BlockSpec((64, 2), lambda i: (0, 0)),
            pl.BlockSpec((64, 2), lambda i: (0, 0)),
            pl.BlockSpec((1, 2), lambda i: (0, 0)),
        ],
        out_specs=[
            pl.BlockSpec((_BLK, 2), lambda i: (i, 0)),
            pl.BlockSpec((_BLK, 1), lambda i: (i, 0)),
            pl.BlockSpec((_BLK, 1), lambda i: (i, 0)),
        ],
        out_shape=[
            jax.ShapeDtypeStruct((npad, 2), jnp.float32),
            jax.ShapeDtypeStruct((npad, 1), jnp.float32),
            jax.ShapeDtypeStruct((npad, 1), jnp.float32),
        ],
    )(h1, qa, qb, dis, W20, W21, b2, W30, W31, b3)


def _tc3_body(nreal, h3a_ref, o00_ref, o01_ref, o10_ref, o11_ref,
              dis_ref, wg_ref, bg_ref, out_ref, acc_ref):
    i = pl.program_id(0)
    ng = pl.num_programs(0)

    @pl.when(i == 0)
    def _():
        acc_ref[0] = -1e30
        acc_ref[1] = 0.0
        acc_ref[2] = 0.0
        acc_ref[3] = 0.0

    q20 = o00_ref[...] + o01_ref[...]
    q21 = o10_ref[...] + o11_ref[...]
    dis = dis_ref[...]
    h3a = h3a_ref[...]
    h30 = h3a[:, 0:1] - dis * q20
    h31 = h3a[:, 1:2] - dis * q21
    wg = wg_ref[...]
    l = h30 * wg[0, 0] + h31 * wg[0, 1] + bg_ref[0, 0]
    rowid = lax.broadcasted_iota(jnp.int32, l.shape, 0) + i * _BLK
    lm = jnp.where(rowid < nreal, l, -1e30)
    m_old = acc_ref[0]
    m_new = jnp.maximum(m_old, jnp.max(lm))
    e = jnp.exp(lm - m_new)
    scale = jnp.exp(m_old - m_new)
    s_new = acc_ref[1] * scale + jnp.sum(e)
    v0 = acc_ref[2] * scale + jnp.sum(e * h30)
    v1 = acc_ref[3] * scale + jnp.sum(e * h31)
    acc_ref[0] = m_new
    acc_ref[1] = s_new
    acc_ref[2] = v0
    acc_ref[3] = v1

    @pl.when(i == ng - 1)
    def _():
        z0 = acc_ref[2] / acc_ref[1]
        z1 = acc_ref[3] / acc_ref[1]
        mz = jnp.maximum(z0, z1)
        lse = mz + jnp.log(jnp.exp(z0 - mz) + jnp.exp(z1 - mz))
        out_ref[...] = jnp.stack([z0 - lse, z1 - lse]).reshape(1, 2)


def _tc3(h3a, o00, o01, o10, o11, dis, Wg, bg, npad, nreal):
    grid = npad // _BLK
    return pl.pallas_call(
        functools.partial(_tc3_body, nreal),
        grid=(grid,),
        in_specs=[
            pl.BlockSpec((_BLK, 2), lambda i: (i, 0)),
            pl.BlockSpec((_BLK, 1), lambda i: (i, 0)),
            pl.BlockSpec((_BLK, 1), lambda i: (i, 0)),
            pl.BlockSpec((_BLK, 1), lambda i: (i, 0)),
            pl.BlockSpec((_BLK, 1), lambda i: (i, 0)),
            pl.BlockSpec((_BLK, 1), lambda i: (i, 0)),
            pl.BlockSpec((1, 2), lambda i: (0, 0)),
            pl.BlockSpec((1, 1), lambda i: (0, 0)),
        ],
        out_specs=pl.BlockSpec((1, 2), lambda i: (0, 0)),
        out_shape=jax.ShapeDtypeStruct((1, 2), jnp.float32),
        scratch_shapes=[pltpu.SMEM((4,), jnp.float32)],
    )(h3a, o00, o01, o10, o11, dis, Wg, bg)


def kernel(x, edge_index, attr, W1, b1, W2, b2, W3, b3, Wg, bg):
    n = x.shape[0]
    e = edge_index.shape[1]
    npad = -(-n // 2048) * 2048
    rows = -(-(-(-e // 128)) // 32) * 32
    epad = rows * 128

    src2d = jnp.pad(edge_index[0], (0, epad - e)).reshape(rows, 128)
    dst2d = jnp.pad(edge_index[1], (0, epad - e)).reshape(rows, 128)
    attr2d = jnp.pad(attr, (0, epad - e)).reshape(rows, 128)
    xp = jnp.pad(x, ((0, npad - n), (0, 0)))

    d0, d1 = _make_sc_degree(rows, npad, 2)(src2d, attr2d)
    h1, ga, gb, dis = _tc1(xp, d0.reshape(npad, 1), d1.reshape(npad, 1),
                           W1[0], b1.reshape(1, 32), npad)
    g1f = jnp.concatenate([ga, gb], axis=0)
    qa, qb = _make_sc_q1(rows, npad, 4)(src2d, dst2d, attr2d, g1f)
    h3a, g20, g21 = _tc2(h1, qa, qb, dis, W2[0], W2[1], b2.reshape(1, 64),
                         W3[0], W3[1], b3.reshape(1, 2), npad)
    o00, o01, o10, o11 = _make_sc_q2(rows, npad, 2)(
        src2d, dst2d, attr2d, g20.reshape(npad), g21.reshape(npad))
    out = _tc3(h3a, o00.reshape(npad, 1), o01.reshape(npad, 1),
               o10.reshape(npad, 1), o11.reshape(npad, 1),
               dis, Wg.reshape(1, 2), bg.reshape(1, 1), npad, n)
    return out


# speedup vs baseline: 49.5936x; 2.0839x over previous
import functools

import jax
import jax.numpy as jnp
from jax import lax
from jax.experimental import pallas as pl
from jax.experimental.pallas import tpu as pltpu
from jax.experimental.pallas import tpu_sc as plsc

_LRELU_SLOPE = 0.01
_MB = 8
_NBUF = 4


def _lrelu(x):
    return jnp.where(x >= 0, x, _LRELU_SLOPE * x)


def _zero_vmem_1d(zb_v, n):
    def body(i, _):
        zb_v[pl.ds(i * 16, 16)] = jnp.zeros((16,), jnp.float32)
        return 0
    lax.fori_loop(0, n // 16, body, 0)


def _zero_vmem_rows(zb_v, n):
    def body(i, _):
        zb_v[i, :] = jnp.zeros((16,), jnp.float32)
        return 0
    lax.fori_loop(0, n, body, 0)


def _run_pipeline(nmac, stage, fire, process, drain):
    assert nmac % _NBUF == 0 and nmac >= 2 * _NBUF
    stage(0, 0)
    stage(1, 1)
    fire(0, 0)

    for m in range(_NBUF):
        fire(m + 1, (m + 1) % _NBUF)
        process(m, m % _NBUF)
        if m + 2 < nmac:
            sp = (m + 2) % _NBUF
            if m >= 2:
                drain(sp)
            stage(m + 2, sp)

    def group(u, _):
        for p in range(_NBUF):
            m = _NBUF + u * _NBUF + p
            fire(m + 1, (p + 1) % _NBUF)
            process(m, p)
            drain((p + 2) % _NBUF)
            stage(m + 2, (p + 2) % _NBUF)
        return 0

    lax.fori_loop(0, (nmac - 2 * _NBUF) // _NBUF, group, 0)

    for m in range(nmac - _NBUF, nmac):
        if m + 1 < nmac:
            fire(m + 1, (m + 1) % _NBUF)
        process(m, m % _NBUF)
        if m + 2 < nmac:
            sp = (m + 2) % _NBUF
            drain(sp)
            stage(m + 2, sp)

    for p in range(_NBUF):
        drain(p)


def _make_sc_degree(rows, npad, mb):
    rpw = rows // 32
    nmac = rpw // mb
    zb = npad // 128
    stripe = npad // 16
    mesh = plsc.VectorSubcoreMesh(core_axis_name="c", subcore_axis_name="s")

    scratch = [pltpu.VMEM_SHARED((npad,), jnp.float32),
               pltpu.VMEM((zb,), jnp.float32)]
    scratch += [pltpu.VMEM((mb, 128), jnp.int32) for _ in range(_NBUF)]
    scratch += [pltpu.VMEM((mb, 128), jnp.float32) for _ in range(_NBUF)]
    scratch += [pltpu.SemaphoreType.DMA for _ in range(2 * _NBUF)]

    @functools.partial(
        pl.kernel,
        mesh=mesh,
        compiler_params=pltpu.CompilerParams(use_tc_tiling_on_sc=False),
        out_type=[jax.ShapeDtypeStruct((npad,), jnp.float32)] * 2,
        scratch_types=scratch,
    )
    def k(src_hbm, attr_hbm, d0_hbm, d1_hbm, deg_sp, zb_v, *bufs):
        srcb = bufs[0:_NBUF]
        attrb = bufs[_NBUF:2 * _NBUF]
        lsem = bufs[2 * _NBUF:3 * _NBUF]
        ssem = bufs[3 * _NBUF:4 * _NBUF]
        c = lax.axis_index("c")
        s = lax.axis_index("s")
        wid = c * 16 + s

        _zero_vmem_1d(zb_v, zb)
        off = s * stripe
        for r in range(8):
            pltpu.sync_copy(zb_v, deg_sp.at[pl.ds(off + r * zb, zb)])
        plsc.subcore_barrier()

        base = wid * rpw

        def stage(m, p):
            r0 = base + m * mb
            pltpu.async_copy(src_hbm.at[pl.ds(r0, mb)], srcb[p], lsem[p])
            pltpu.async_copy(attr_hbm.at[pl.ds(r0, mb)], attrb[p], lsem[p])

        def fire(m, p):
            pass

        def process(m, p):
            pltpu.make_async_copy(src_hbm.at[pl.ds(0, mb)], srcb[p],
                                  lsem[p]).wait()
            pltpu.make_async_copy(attr_hbm.at[pl.ds(0, mb)], attrb[p],
                                  lsem[p]).wait()
            for j in range(mb):
                pltpu.async_copy(attrb[p].at[j], deg_sp.at[srcb[p].at[j]],
                                 ssem[p], add=True)

        def drain(p):
            for j in range(mb):
                pltpu.make_async_copy(attrb[p].at[0],
                                      deg_sp.at[srcb[p].at[0]],
                                      ssem[p]).wait()

        _run_pipeline(nmac, stage, fire, process, drain)
        plsc.subcore_barrier()

        @pl.when(c == 0)
        def _():
            pltpu.sync_copy(deg_sp.at[pl.ds(off, stripe)],
                            d0_hbm.at[pl.ds(off, stripe)])

        @pl.when(c == 1)
        def _():
            pltpu.sync_copy(deg_sp.at[pl.ds(off, stripe)],
                            d1_hbm.at[pl.ds(off, stripe)])

    return k


def _make_sc_q1(rows, npad, mb):
    rpw = rows // 16
    nmac = rpw // mb
    zb = npad // 128
    stripe = npad // 16
    mesh = plsc.VectorSubcoreMesh(core_axis_name="c", subcore_axis_name="s")

    scratch = [pltpu.VMEM_SHARED((npad, 16), jnp.float32),
               pltpu.VMEM((128, 16), jnp.float32)]
    scratch += [pltpu.VMEM((mb, 128), jnp.int32) for _ in range(_NBUF)]
    scratch += [pltpu.VMEM((mb, 128), jnp.int32) for _ in range(_NBUF)]
    scratch += [pltpu.VMEM((mb, 128), jnp.float32) for _ in range(_NBUF)]
    scratch += [pltpu.VMEM((mb * 128, 16), jnp.float32)
                for _ in range(_NBUF)]
    scratch += [pltpu.SemaphoreType.DMA for _ in range(3 * _NBUF)]

    @functools.partial(
        pl.kernel,
        mesh=mesh,
        compiler_params=pltpu.CompilerParams(use_tc_tiling_on_sc=False),
        out_type=[jax.ShapeDtypeStruct((npad, 16), jnp.float32)] * 2,
        scratch_types=scratch,
    )
    def k(src_hbm, dst_hbm, attr_hbm, g1f_hbm, qa_hbm, qb_hbm,
          q_sp, zb_v, *bufs):
        srcb = bufs[0:_NBUF]
        dstb = bufs[_NBUF:2 * _NBUF]
        attrb = bufs[2 * _NBUF:3 * _NBUF]
        rowsb = bufs[3 * _NBUF:4 * _NBUF]
        lsem = bufs[4 * _NBUF:5 * _NBUF]
        gsem = bufs[5 * _NBUF:6 * _NBUF]
        ssem = bufs[6 * _NBUF:7 * _NBUF]
        c = lax.axis_index("c")
        s = lax.axis_index("s")

        _zero_vmem_rows(zb_v, 128)
        off = s * stripe
        for r in range(stripe // 128):
            pltpu.sync_copy(zb_v, q_sp.at[pl.ds(off + r * 128, 128)])
        plsc.subcore_barrier()

        base = s * rpw
        coff = jnp.full((16,), c * npad, dtype=jnp.int32)

        def stage(m, p):
            r0 = base + m * mb
            pltpu.async_copy(src_hbm.at[pl.ds(r0, mb)], srcb[p], lsem[p])
            pltpu.async_copy(dst_hbm.at[pl.ds(r0, mb)], dstb[p], lsem[p])
            pltpu.async_copy(attr_hbm.at[pl.ds(r0, mb)], attrb[p], lsem[p])

        def fire(m, p):
            for _ in range(3):
                pltpu.make_async_copy(attr_hbm.at[pl.ds(0, mb)], attrb[p],
                                      lsem[p]).wait()
            for j in range(mb):
                for g in range(8):
                    sl = pl.ds(g * 16, 16)
                    srcb[p][j, sl] = srcb[p][j, sl] + coff
            for j in range(mb):
                pltpu.async_copy(g1f_hbm.at[srcb[p].at[j]],
                                 rowsb[p].at[pl.ds(j * 128, 128)], gsem[p])

        def process(m, p):
            for _ in range(mb):
                pltpu.make_async_copy(g1f_hbm.at[srcb[p].at[0]],
                                      rowsb[p].at[pl.ds(0, 128)],
                                      gsem[p]).wait()

            def gbody(g, _):
                j = g >> 3
                av = attrb[p][j, pl.ds((g & 7) * 16, 16)]
                for li in range(16):
                    i = g * 16 + li
                    rowsb[p][i, :] = rowsb[p][i, :] * av[li]
                return 0

            lax.fori_loop(0, mb * 8, gbody, 0)
            for j in range(mb):
                pltpu.async_copy(rowsb[p].at[pl.ds(j * 128, 128)],
                                 q_sp.at[dstb[p].at[j]], ssem[p], add=True)

        def drain(p):
            for _ in range(mb):
                pltpu.make_async_copy(rowsb[p].at[pl.ds(0, 128)],
                                      q_sp.at[dstb[p].at[0]],
                                      ssem[p]).wait()

        _run_pipeline(nmac, stage, fire, process, drain)
        plsc.subcore_barrier()

        @pl.when(c == 0)
        def _():
            pltpu.sync_copy(q_sp.at[pl.ds(off, stripe)],
                            qa_hbm.at[pl.ds(off, stripe)])

        @pl.when(c == 1)
        def _():
            pltpu.sync_copy(q_sp.at[pl.ds(off, stripe)],
                            qb_hbm.at[pl.ds(off, stripe)])

    return k


def _make_sc_q2(rows, npad, mb):
    rpw = rows // 32
    nmac = rpw // mb
    zb = npad // 128
    stripe = npad // 16
    mesh = plsc.VectorSubcoreMesh(core_axis_name="c", subcore_axis_name="s")

    scratch = [pltpu.VMEM_SHARED((npad,), jnp.float32) for _ in range(4)]
    scratch += [pltpu.VMEM((zb,), jnp.float32)]
    scratch += [pltpu.VMEM((mb, 128), jnp.int32) for _ in range(_NBUF)]
    scratch += [pltpu.VMEM((mb, 128), jnp.int32) for _ in range(_NBUF)]
    scratch += [pltpu.VMEM((mb, 128), jnp.float32) for _ in range(_NBUF)]
    scratch += [pltpu.VMEM((mb * 128,), jnp.float32) for _ in range(_NBUF)]
    scratch += [pltpu.VMEM((mb * 128,), jnp.float32) for _ in range(_NBUF)]
    scratch += [pltpu.SemaphoreType.DMA for _ in range(3 * _NBUF)]

    @functools.partial(
        pl.kernel,
        mesh=mesh,
        compiler_params=pltpu.CompilerParams(use_tc_tiling_on_sc=False),
        out_type=[jax.ShapeDtypeStruct((npad,), jnp.float32)] * 4,
        scratch_types=scratch,
    )
    def k(src_hbm, dst_hbm, attr_hbm, g20_hbm, g21_hbm,
          o00_hbm, o01_hbm, o10_hbm, o11_hbm,
          g0_sp, g1_sp, q0_sp, q1_sp, zb_v, *bufs):
        srcb = bufs[0:_NBUF]
        dstb = bufs[_NBUF:2 * _NBUF]
        attrb = bufs[2 * _NBUF:3 * _NBUF]
        r0b = bufs[3 * _NBUF:4 * _NBUF]
        r1b = bufs[4 * _NBUF:5 * _NBUF]
        lsem = bufs[5 * _NBUF:6 * _NBUF]
        gsem = bufs[6 * _NBUF:7 * _NBUF]
        ssem = bufs[7 * _NBUF:8 * _NBUF]
        c = lax.axis_index("c")
        s = lax.axis_index("s")
        wid = c * 16 + s

        off = s * stripe
        pltpu.sync_copy(g20_hbm.at[pl.ds(off, stripe)],
                        g0_sp.at[pl.ds(off, stripe)])
        pltpu.sync_copy(g21_hbm.at[pl.ds(off, stripe)],
                        g1_sp.at[pl.ds(off, stripe)])
        _zero_vmem_1d(zb_v, zb)
        for r in range(8):
            pltpu.sync_copy(zb_v, q0_sp.at[pl.ds(off + r * zb, zb)])
            pltpu.sync_copy(zb_v, q1_sp.at[pl.ds(off + r * zb, zb)])
        plsc.subcore_barrier()

        base = wid * rpw

        def stage(m, p):
            r0 = base + m * mb
            pltpu.async_copy(src_hbm.at[pl.ds(r0, mb)], srcb[p], lsem[p])
            pltpu.async_copy(dst_hbm.at[pl.ds(r0, mb)], dstb[p], lsem[p])
            pltpu.async_copy(attr_hbm.at[pl.ds(r0, mb)], attrb[p], lsem[p])

        def fire(m, p):
            for _ in range(3):
                pltpu.make_async_copy(attr_hbm.at[pl.ds(0, mb)], attrb[p],
                                      lsem[p]).wait()
            for j in range(mb):
                sl = pl.ds(j * 128, 128)
                pltpu.async_copy(g0_sp.at[srcb[p].at[j]], r0b[p].at[sl],
                                 gsem[p])
                pltpu.async_copy(g1_sp.at[srcb[p].at[j]], r1b[p].at[sl],
                                 gsem[p])

        def process(m, p):
            for _ in range(2 * mb):
                pltpu.make_async_copy(g0_sp.at[srcb[p].at[0]],
                                      r0b[p].at[pl.ds(0, 128)],
                                      gsem[p]).wait()

            def gbody(g, _):
                j = g >> 3
                sl16 = pl.ds(g * 16, 16)
                av = attrb[p][j, pl.ds((g & 7) * 16, 16)]
                r0b[p][sl16] = r0b[p][sl16] * av
                r1b[p][sl16] = r1b[p][sl16] * av
                return 0

            lax.fori_loop(0, mb * 8, gbody, 0)
            for j in range(mb):
                sl = pl.ds(j * 128, 128)
                pltpu.async_copy(r0b[p].at[sl], q0_sp.at[dstb[p].at[j]],
                                 ssem[p], add=True)
                pltpu.async_copy(r1b[p].at[sl], q1_sp.at[dstb[p].at[j]],
                                 ssem[p], add=True)

        def drain(p):
            for _ in range(2 * mb):
                pltpu.make_async_copy(r0b[p].at[pl.ds(0, 128)],
                                      q0_sp.at[dstb[p].at[0]],
                                      ssem[p]).wait()

        _run_pipeline(nmac, stage, fire, process, drain)
        plsc.subcore_barrier()

        @pl.when(c == 0)
        def _():
            pltpu.sync_copy(q0_sp.at[pl.ds(off, stripe)],
                            o00_hbm.at[pl.ds(off, stripe)])
            pltpu.sync_copy(q1_sp.at[pl.ds(off, stripe)],
                            o10_hbm.at[pl.ds(off, stripe)])

        @pl.when(c == 1)
        def _():
            pltpu.sync_copy(q0_sp.at[pl.ds(off, stripe)],
                            o01_hbm.at[pl.ds(off, stripe)])
            pltpu.sync_copy(q1_sp.at[pl.ds(off, stripe)],
                            o11_hbm.at[pl.ds(off, stripe)])

    return k


_BLK = 2048


def _tc1_body(x_ref, d0_ref, d1_ref, w_ref, b_ref,
              h1_ref, ga_ref, gb_ref, dis_ref):
    deg = d0_ref[...] + d1_ref[...]
    dis = jnp.where(deg > 0, lax.rsqrt(deg), 0.0)
    h = jnp.dot(x_ref[...], w_ref[...],
                preferred_element_type=jnp.float32) + b_ref[...]
    h1 = _lrelu(h)
    h1_ref[...] = h1
    g = dis * h1
    ga_ref[...] = g[:, :16]
    gb_ref[...] = g[:, 16:]
    dis_ref[...] = dis


def _tc1(xp, d0, d1, W10, b1, npad):
    grid = npad // _BLK
    return pl.pallas_call(
        _tc1_body,
        grid=(grid,),
        in_specs=[
            pl.BlockSpec((_BLK, 20), lambda i: (i, 0)),
            pl.BlockSpec((_BLK, 1), lambda i: (i, 0)),
            pl.BlockSpec((_BLK, 1), lambda i: (i, 0)),
            pl.BlockSpec((20, 32), lambda i: (0, 0)),
            pl.BlockSpec((1, 32), lambda i: (0, 0)),
        ],
        out_specs=[
            pl.BlockSpec((_BLK, 32), lambda i: (i, 0)),
            pl.BlockSpec((_BLK, 16), lambda i: (i, 0)),
            pl.BlockSpec((_BLK, 16), lambda i: (i, 0)),
            pl.BlockSpec((_BLK, 1), lambda i: (i, 0)),
        ],
        out_shape=[
            jax.ShapeDtypeStruct((npad, 32), jnp.float32),
            jax.ShapeDtypeStruct((npad, 16), jnp.float32),
            jax.ShapeDtypeStruct((npad, 16), jnp.float32),
            jax.ShapeDtypeStruct((npad, 1), jnp.float32),
        ],
    )(xp, d0, d1, W10, b1)


def _tc2_body(h1_ref, qa_ref, qb_ref, dis_ref,
              w20_ref, w21_ref, b2_ref, w30_ref, w31_ref, b3_ref,
              h3a_ref, g20_ref, g21_ref):
    dis = dis_ref[...]
    q1 = jnp.concatenate([qa_ref[...], qb_ref[...]], axis=1)
    p1 = -dis * q1
    h2 = _lrelu(
        jnp.dot(h1_ref[...], w20_ref[...],
                preferred_element_type=jnp.float32)
        + jnp.dot(p1, w21_ref[...], preferred_element_type=jnp.float32)
        + b2_ref[...])
    t = jnp.dot(h2, w31_ref[...], preferred_element_type=jnp.float32)
    g2 = dis * t
    g20_ref[...] = g2[:, 0:1]
    g21_ref[...] = g2[:, 1:2]
    h3a_ref[...] = jnp.dot(h2, w30_ref[...],
                           preferred_element_type=jnp.float32) + b3_ref[...]


def _tc2(h1, qa, qb, dis, W20, W21, b2, W30, W31, b3, npad):
    grid = npad // _BLK
    return pl.pallas_call(
        _tc2_body,
        grid=(grid,),
        in_specs=[
            pl.BlockSpec((_BLK, 32), lambda i: (i, 0)),
            pl.BlockSpec((_BLK, 16), lambda i: (i, 0)),
            pl.BlockSpec((_BLK, 16), lambda i: (i, 0)),
            pl.BlockSpec((_BLK, 1), lambda i: (i, 0)),
            pl.BlockSpec((32, 64), lambda i: (0, 0)),
            pl.BlockSpec((32, 64), lambda i: (0, 0)),
            pl.BlockSpec((1, 64), lambda i: (0, 0)),
            pl.BlockSpec((64, 2), lambda i: (0, 0)),
            pl.BlockSpec((64, 2), lambda i: (0, 0)),
            pl.BlockSpec((1, 2), lambda i: (0, 0)),
        ],
        out_specs=[
            pl.BlockSpec((_BLK, 2), lambda i: (i, 0)),
            pl.BlockSpec((_BLK, 1), lambda i: (i, 0)),
            pl.BlockSpec((_BLK, 1), lambda i: (i, 0)),
        ],
        out_shape=[
            jax.ShapeDtypeStruct((npad, 2), jnp.float32),
            jax.ShapeDtypeStruct((npad, 1), jnp.float32),
            jax.ShapeDtypeStruct((npad, 1), jnp.float32),
        ],
    )(h1, qa, qb, dis, W20, W21, b2, W30, W31, b3)


def _tc3_body(nreal, h3a_ref, o00_ref, o01_ref, o10_ref, o11_ref,
              dis_ref, wg_ref, bg_ref, out_ref, acc_ref):
    i = pl.program_id(0)
    ng = pl.num_programs(0)

    @pl.when(i == 0)
    def _():
        acc_ref[0] = -1e30
        acc_ref[1] = 0.0
        acc_ref[2] = 0.0
        acc_ref[3] = 0.0

    q20 = o00_ref[...] + o01_ref[...]
    q21 = o10_ref[...] + o11_ref[...]
    dis = dis_ref[...]
    h3a = h3a_ref[...]
    h30 = h3a[:, 0:1] - dis * q20
    h31 = h3a[:, 1:2] - dis * q21
    wg = wg_ref[...]
    l = h30 * wg[0, 0] + h31 * wg[0, 1] + bg_ref[0, 0]
    rowid = lax.broadcasted_iota(jnp.int32, l.shape, 0) + i * _BLK
    lm = jnp.where(rowid < nreal, l, -1e30)
    m_old = acc_ref[0]
    m_new = jnp.maximum(m_old, jnp.max(lm))
    e = jnp.exp(lm - m_new)
    scale = jnp.exp(m_old - m_new)
    s_new = acc_ref[1] * scale + jnp.sum(e)
    v0 = acc_ref[2] * scale + jnp.sum(e * h30)
    v1 = acc_ref[3] * scale + jnp.sum(e * h31)
    acc_ref[0] = m_new
    acc_ref[1] = s_new
    acc_ref[2] = v0
    acc_ref[3] = v1

    @pl.when(i == ng - 1)
    def _():
        z0 = acc_ref[2] / acc_ref[1]
        z1 = acc_ref[3] / acc_ref[1]
        mz = jnp.maximum(z0, z1)
        lse = mz + jnp.log(jnp.exp(z0 - mz) + jnp.exp(z1 - mz))
        out_ref[...] = jnp.stack([z0 - lse, z1 - lse]).reshape(1, 2)


def _tc3(h3a, o00, o01, o10, o11, dis, Wg, bg, npad, nreal):
    grid = npad // _BLK
    return pl.pallas_call(
        functools.partial(_tc3_body, nreal),
        grid=(grid,),
        in_specs=[
            pl.BlockSpec((_BLK, 2), lambda i: (i, 0)),
            pl.BlockSpec((_BLK, 1), lambda i: (i, 0)),
            pl.BlockSpec((_BLK, 1), lambda i: (i, 0)),
            pl.BlockSpec((_BLK, 1), lambda i: (i, 0)),
            pl.BlockSpec((_BLK, 1), lambda i: (i, 0)),
            pl.BlockSpec((_BLK, 1), lambda i: (i, 0)),
            pl.BlockSpec((1, 2), lambda i: (0, 0)),
            pl.BlockSpec((1, 1), lambda i: (0, 0)),
        ],
        out_specs=pl.BlockSpec((1, 2), lambda i: (0, 0)),
        out_shape=jax.ShapeDtypeStruct((1, 2), jnp.float32),
        scratch_shapes=[pltpu.SMEM((4,), jnp.float32)],
    )(h3a, o00, o01, o10, o11, dis, Wg, bg)


def kernel(x, edge_index, attr, W1, b1, W2, b2, W3, b3, Wg, bg):
    n = x.shape[0]
    e = edge_index.shape[1]
    npad = -(-n // 2048) * 2048
    rows128 = -(-e // 128)
    rows = -(-rows128 // 1024) * 1024
    epad = rows * 128

    src2d = jnp.pad(edge_index[0], (0, epad - e)).reshape(rows, 128)
    dst2d = jnp.pad(edge_index[1], (0, epad - e)).reshape(rows, 128)
    attr2d = jnp.pad(attr, (0, epad - e)).reshape(rows, 128)
    xp = jnp.pad(x, ((0, npad - n), (0, 0)))

    d0, d1 = _make_sc_degree(rows, npad, 8)(src2d, attr2d)
    h1, ga, gb, dis = _tc1(xp, d0.reshape(npad, 1), d1.reshape(npad, 1),
                           W1[0], b1.reshape(1, 32), npad)
    g1f = jnp.concatenate([ga, gb], axis=0)
    qa, qb = _make_sc_q1(rows, npad, 2)(src2d, dst2d, attr2d, g1f)
    h3a, g20, g21 = _tc2(h1, qa, qb, dis, W2[0], W2[1], b2.reshape(1, 64),
                         W3[0], W3[1], b3.reshape(1, 2), npad)
    o00, o01, o10, o11 = _make_sc_q2(rows, npad, 4)(
        src2d, dst2d, attr2d, g20.reshape(npad), g21.reshape(npad))
    out = _tc3(h3a, o00.reshape(npad, 1), o01.reshape(npad, 1),
               o10.reshape(npad, 1), o11.reshape(npad, 1),
               dis, Wg.reshape(1, 2), bg.reshape(1, 1), npad, n)
    return out


# speedup vs baseline: 49.9929x; 1.0081x over previous
import functools

import jax
import jax.numpy as jnp
from jax import lax
from jax.experimental import pallas as pl
from jax.experimental.pallas import tpu as pltpu
from jax.experimental.pallas import tpu_sc as plsc

_LRELU_SLOPE = 0.01
_MB = 8
_NBUF = 4


def _lrelu(x):
    return jnp.where(x >= 0, x, _LRELU_SLOPE * x)


def _zero_vmem_1d(zb_v, n):
    def body(i, _):
        zb_v[pl.ds(i * 16, 16)] = jnp.zeros((16,), jnp.float32)
        return 0
    lax.fori_loop(0, n // 16, body, 0)


def _zero_vmem_rows(zb_v, n):
    def body(i, _):
        zb_v[i, :] = jnp.zeros((16,), jnp.float32)
        return 0
    lax.fori_loop(0, n, body, 0)


def _run_pipeline(nmac, nbuf, ahead, stage, fire, process, drain):
    assert nmac % nbuf == 0 and nmac >= 2 * nbuf and nbuf >= ahead + 2
    for q in range(ahead + 1):
        stage(q, q % nbuf)
    for q in range(ahead):
        fire(q, q % nbuf)

    def emit(m, p, mstat):
        if mstat + ahead < nmac:
            fire(m + ahead, (p + ahead) % nbuf)
        process(m, p)
        if mstat + ahead + 1 < nmac:
            sp = (p + ahead + 1) % nbuf
            if mstat >= nbuf - ahead - 1:
                drain(sp)
            stage(m + ahead + 1, sp)

    for m in range(nbuf):
        emit(m, m % nbuf, m)

    def group(u, _):
        for p in range(nbuf):
            m = nbuf + u * nbuf + p
            emit(m, p, nbuf)
        return 0

    lax.fori_loop(0, (nmac - 2 * nbuf) // nbuf, group, 0)

    for m in range(nmac - nbuf, nmac):
        emit(m, m % nbuf, m)

    for p in range(nbuf):
        drain(p)


def _make_sc_degree(rows, npad, mb):
    rpw = rows // 32
    nmac = rpw // mb
    zb = npad // 128
    stripe = npad // 16
    nb, ahead = 4, 1
    mesh = plsc.VectorSubcoreMesh(core_axis_name="c", subcore_axis_name="s")

    scratch = [pltpu.VMEM_SHARED((npad,), jnp.float32),
               pltpu.VMEM((zb,), jnp.float32)]
    scratch += [pltpu.VMEM((mb, 128), jnp.int32) for _ in range(nb)]
    scratch += [pltpu.VMEM((mb, 128), jnp.float32) for _ in range(nb)]
    scratch += [pltpu.SemaphoreType.DMA for _ in range(2 * nb)]

    @functools.partial(
        pl.kernel,
        mesh=mesh,
        compiler_params=pltpu.CompilerParams(use_tc_tiling_on_sc=False),
        out_type=[jax.ShapeDtypeStruct((npad,), jnp.float32)] * 2,
        scratch_types=scratch,
    )
    def k(src_hbm, attr_hbm, d0_hbm, d1_hbm, deg_sp, zb_v, *bufs):
        srcb = bufs[0:nb]
        attrb = bufs[nb:2 * nb]
        lsem = bufs[2 * nb:3 * nb]
        ssem = bufs[3 * nb:4 * nb]
        c = lax.axis_index("c")
        s = lax.axis_index("s")
        wid = c * 16 + s

        _zero_vmem_1d(zb_v, zb)
        off = s * stripe
        for r in range(8):
            pltpu.sync_copy(zb_v, deg_sp.at[pl.ds(off + r * zb, zb)])
        plsc.subcore_barrier()

        base = wid * rpw

        def stage(m, p):
            r0 = base + m * mb
            pltpu.async_copy(src_hbm.at[pl.ds(r0, mb)], srcb[p], lsem[p])
            pltpu.async_copy(attr_hbm.at[pl.ds(r0, mb)], attrb[p], lsem[p])

        def fire(m, p):
            pass

        def process(m, p):
            pltpu.make_async_copy(src_hbm.at[pl.ds(0, mb)], srcb[p],
                                  lsem[p]).wait()
            pltpu.make_async_copy(attr_hbm.at[pl.ds(0, mb)], attrb[p],
                                  lsem[p]).wait()
            for j in range(mb):
                pltpu.async_copy(attrb[p].at[j], deg_sp.at[srcb[p].at[j]],
                                 ssem[p], add=True)

        def drain(p):
            for j in range(mb):
                pltpu.make_async_copy(attrb[p].at[0],
                                      deg_sp.at[srcb[p].at[0]],
                                      ssem[p]).wait()

        _run_pipeline(nmac, nb, ahead, stage, fire, process, drain)
        plsc.subcore_barrier()

        @pl.when(c == 0)
        def _():
            pltpu.sync_copy(deg_sp.at[pl.ds(off, stripe)],
                            d0_hbm.at[pl.ds(off, stripe)])

        @pl.when(c == 1)
        def _():
            pltpu.sync_copy(deg_sp.at[pl.ds(off, stripe)],
                            d1_hbm.at[pl.ds(off, stripe)])

    return k


def _make_sc_q1(rows, npad, mb):
    rpw = rows // 16
    nmac = rpw // mb
    zb = npad // 128
    stripe = npad // 16
    nb, ahead = 5, 2
    mesh = plsc.VectorSubcoreMesh(core_axis_name="c", subcore_axis_name="s")

    scratch = [pltpu.VMEM_SHARED((npad, 16), jnp.float32),
               pltpu.VMEM((128, 16), jnp.float32)]
    scratch += [pltpu.VMEM((mb, 128), jnp.int32) for _ in range(nb)]
    scratch += [pltpu.VMEM((mb, 128), jnp.int32) for _ in range(nb)]
    scratch += [pltpu.VMEM((mb, 128), jnp.float32) for _ in range(nb)]
    scratch += [pltpu.VMEM((mb * 128, 16), jnp.float32)
                for _ in range(nb)]
    scratch += [pltpu.SemaphoreType.DMA for _ in range(3 * nb)]

    @functools.partial(
        pl.kernel,
        mesh=mesh,
        compiler_params=pltpu.CompilerParams(use_tc_tiling_on_sc=False),
        out_type=[jax.ShapeDtypeStruct((npad, 16), jnp.float32)] * 2,
        scratch_types=scratch,
    )
    def k(src_hbm, dst_hbm, attr_hbm, g1f_hbm, qa_hbm, qb_hbm,
          q_sp, zb_v, *bufs):
        srcb = bufs[0:nb]
        dstb = bufs[nb:2 * nb]
        attrb = bufs[2 * nb:3 * nb]
        rowsb = bufs[3 * nb:4 * nb]
        lsem = bufs[4 * nb:5 * nb]
        gsem = bufs[5 * nb:6 * nb]
        ssem = bufs[6 * nb:7 * nb]
        c = lax.axis_index("c")
        s = lax.axis_index("s")

        _zero_vmem_rows(zb_v, 128)
        off = s * stripe
        for r in range(stripe // 128):
            pltpu.sync_copy(zb_v, q_sp.at[pl.ds(off + r * 128, 128)])
        plsc.subcore_barrier()

        base = s * rpw
        coff = jnp.full((16,), c * npad, dtype=jnp.int32)

        def stage(m, p):
            r0 = base + m * mb
            pltpu.async_copy(src_hbm.at[pl.ds(r0, mb)], srcb[p], lsem[p])
            pltpu.async_copy(dst_hbm.at[pl.ds(r0, mb)], dstb[p], lsem[p])
            pltpu.async_copy(attr_hbm.at[pl.ds(r0, mb)], attrb[p], lsem[p])

        def fire(m, p):
            for _ in range(3):
                pltpu.make_async_copy(attr_hbm.at[pl.ds(0, mb)], attrb[p],
                                      lsem[p]).wait()
            for j in range(mb):
                for g in range(8):
                    sl = pl.ds(g * 16, 16)
                    srcb[p][j, sl] = srcb[p][j, sl] + coff
            for j in range(mb):
                pltpu.async_copy(g1f_hbm.at[srcb[p].at[j]],
                                 rowsb[p].at[pl.ds(j * 128, 128)], gsem[p])

        def process(m, p):
            for _ in range(mb):
                pltpu.make_async_copy(g1f_hbm.at[srcb[p].at[0]],
                                      rowsb[p].at[pl.ds(0, 128)],
                                      gsem[p]).wait()

            def gbody(g, _):
                j = g >> 3
                av = attrb[p][j, pl.ds((g & 7) * 16, 16)]
                for li in range(16):
                    i = g * 16 + li
                    rowsb[p][i, :] = rowsb[p][i, :] * av[li]
                return 0

            lax.fori_loop(0, mb * 8, gbody, 0)
            for j in range(mb):
                pltpu.async_copy(rowsb[p].at[pl.ds(j * 128, 128)],
                                 q_sp.at[dstb[p].at[j]], ssem[p], add=True)

        def drain(p):
            for _ in range(mb):
                pltpu.make_async_copy(rowsb[p].at[pl.ds(0, 128)],
                                      q_sp.at[dstb[p].at[0]],
                                      ssem[p]).wait()

        _run_pipeline(nmac, nb, ahead, stage, fire, process, drain)
        plsc.subcore_barrier()

        @pl.when(c == 0)
        def _():
            pltpu.sync_copy(q_sp.at[pl.ds(off, stripe)],
                            qa_hbm.at[pl.ds(off, stripe)])

        @pl.when(c == 1)
        def _():
            pltpu.sync_copy(q_sp.at[pl.ds(off, stripe)],
                            qb_hbm.at[pl.ds(off, stripe)])

    return k


def _make_sc_q2(rows, npad, mb):
    rpw = rows // 32
    nmac = rpw // mb
    zb = npad // 128
    stripe = npad // 16
    nb, ahead = 4, 1
    mesh = plsc.VectorSubcoreMesh(core_axis_name="c", subcore_axis_name="s")

    scratch = [pltpu.VMEM_SHARED((npad,), jnp.float32) for _ in range(4)]
    scratch += [pltpu.VMEM((zb,), jnp.float32)]
    scratch += [pltpu.VMEM((mb, 128), jnp.int32) for _ in range(nb)]
    scratch += [pltpu.VMEM((mb, 128), jnp.int32) for _ in range(nb)]
    scratch += [pltpu.VMEM((mb, 128), jnp.float32) for _ in range(nb)]
    scratch += [pltpu.VMEM((mb * 128,), jnp.float32) for _ in range(nb)]
    scratch += [pltpu.VMEM((mb * 128,), jnp.float32) for _ in range(nb)]
    scratch += [pltpu.SemaphoreType.DMA for _ in range(3 * nb)]

    @functools.partial(
        pl.kernel,
        mesh=mesh,
        compiler_params=pltpu.CompilerParams(use_tc_tiling_on_sc=False),
        out_type=[jax.ShapeDtypeStruct((npad,), jnp.float32)] * 4,
        scratch_types=scratch,
    )
    def k(src_hbm, dst_hbm, attr_hbm, g20_hbm, g21_hbm,
          o00_hbm, o01_hbm, o10_hbm, o11_hbm,
          g0_sp, g1_sp, q0_sp, q1_sp, zb_v, *bufs):
        srcb = bufs[0:nb]
        dstb = bufs[nb:2 * nb]
        attrb = bufs[2 * nb:3 * nb]
        r0b = bufs[3 * nb:4 * nb]
        r1b = bufs[4 * nb:5 * nb]
        lsem = bufs[5 * nb:6 * nb]
        gsem = bufs[6 * nb:7 * nb]
        ssem = bufs[7 * nb:8 * nb]
        c = lax.axis_index("c")
        s = lax.axis_index("s")
        wid = c * 16 + s

        off = s * stripe
        pltpu.sync_copy(g20_hbm.at[pl.ds(off, stripe)],
                        g0_sp.at[pl.ds(off, stripe)])
        pltpu.sync_copy(g21_hbm.at[pl.ds(off, stripe)],
                        g1_sp.at[pl.ds(off, stripe)])
        _zero_vmem_1d(zb_v, zb)
        for r in range(8):
            pltpu.sync_copy(zb_v, q0_sp.at[pl.ds(off + r * zb, zb)])
            pltpu.sync_copy(zb_v, q1_sp.at[pl.ds(off + r * zb, zb)])
        plsc.subcore_barrier()

        base = wid * rpw

        def stage(m, p):
            r0 = base + m * mb
            pltpu.async_copy(src_hbm.at[pl.ds(r0, mb)], srcb[p], lsem[p])
            pltpu.async_copy(dst_hbm.at[pl.ds(r0, mb)], dstb[p], lsem[p])
            pltpu.async_copy(attr_hbm.at[pl.ds(r0, mb)], attrb[p], lsem[p])

        def fire(m, p):
            for _ in range(3):
                pltpu.make_async_copy(attr_hbm.at[pl.ds(0, mb)], attrb[p],
                                      lsem[p]).wait()
            for j in range(mb):
                sl = pl.ds(j * 128, 128)
                pltpu.async_copy(g0_sp.at[srcb[p].at[j]], r0b[p].at[sl],
                                 gsem[p])
                pltpu.async_copy(g1_sp.at[srcb[p].at[j]], r1b[p].at[sl],
                                 gsem[p])

        def process(m, p):
            for _ in range(2 * mb):
                pltpu.make_async_copy(g0_sp.at[srcb[p].at[0]],
                                      r0b[p].at[pl.ds(0, 128)],
                                      gsem[p]).wait()

            def gbody(g, _):
                j = g >> 3
                sl16 = pl.ds(g * 16, 16)
                av = attrb[p][j, pl.ds((g & 7) * 16, 16)]
                r0b[p][sl16] = r0b[p][sl16] * av
                r1b[p][sl16] = r1b[p][sl16] * av
                return 0

            lax.fori_loop(0, mb * 8, gbody, 0)
            for j in range(mb):
                sl = pl.ds(j * 128, 128)
                pltpu.async_copy(r0b[p].at[sl], q0_sp.at[dstb[p].at[j]],
                                 ssem[p], add=True)
                pltpu.async_copy(r1b[p].at[sl], q1_sp.at[dstb[p].at[j]],
                                 ssem[p], add=True)

        def drain(p):
            for _ in range(2 * mb):
                pltpu.make_async_copy(r0b[p].at[pl.ds(0, 128)],
                                      q0_sp.at[dstb[p].at[0]],
                                      ssem[p]).wait()

        _run_pipeline(nmac, nb, ahead, stage, fire, process, drain)
        plsc.subcore_barrier()

        @pl.when(c == 0)
        def _():
            pltpu.sync_copy(q0_sp.at[pl.ds(off, stripe)],
                            o00_hbm.at[pl.ds(off, stripe)])
            pltpu.sync_copy(q1_sp.at[pl.ds(off, stripe)],
                            o10_hbm.at[pl.ds(off, stripe)])

        @pl.when(c == 1)
        def _():
            pltpu.sync_copy(q0_sp.at[pl.ds(off, stripe)],
                            o01_hbm.at[pl.ds(off, stripe)])
            pltpu.sync_copy(q1_sp.at[pl.ds(off, stripe)],
                            o11_hbm.at[pl.ds(off, stripe)])

    return k


_BLK = 2048


def _tc1_body(x_ref, d0_ref, d1_ref, w_ref, b_ref,
              h1_ref, ga_ref, gb_ref, dis_ref):
    deg = d0_ref[...] + d1_ref[...]
    dis = jnp.where(deg > 0, lax.rsqrt(deg), 0.0)
    h = jnp.dot(x_ref[...], w_ref[...],
                preferred_element_type=jnp.float32) + b_ref[...]
    h1 = _lrelu(h)
    h1_ref[...] = h1
    g = dis * h1
    ga_ref[...] = g[:, :16]
    gb_ref[...] = g[:, 16:]
    dis_ref[...] = dis


def _tc1(xp, d0, d1, W10, b1, npad):
    grid = npad // _BLK
    return pl.pallas_call(
        _tc1_body,
        grid=(grid,),
        in_specs=[
            pl.BlockSpec((_BLK, 20), lambda i: (i, 0)),
            pl.BlockSpec((_BLK, 1), lambda i: (i, 0)),
            pl.BlockSpec((_BLK, 1), lambda i: (i, 0)),
            pl.BlockSpec((20, 32), lambda i: (0, 0)),
            pl.BlockSpec((1, 32), lambda i: (0, 0)),
        ],
        out_specs=[
            pl.BlockSpec((_BLK, 32), lambda i: (i, 0)),
            pl.BlockSpec((_BLK, 16), lambda i: (i, 0)),
            pl.BlockSpec((_BLK, 16), lambda i: (i, 0)),
            pl.BlockSpec((_BLK, 1), lambda i: (i, 0)),
        ],
        out_shape=[
            jax.ShapeDtypeStruct((npad, 32), jnp.float32),
            jax.ShapeDtypeStruct((npad, 16), jnp.float32),
            jax.ShapeDtypeStruct((npad, 16), jnp.float32),
            jax.ShapeDtypeStruct((npad, 1), jnp.float32),
        ],
    )(xp, d0, d1, W10, b1)


def _tc2_body(h1_ref, qa_ref, qb_ref, dis_ref,
              w20_ref, w21_ref, b2_ref, w30_ref, w31_ref, b3_ref,
              h3a_ref, g20_ref, g21_ref):
    dis = dis_ref[...]
    q1 = jnp.concatenate([qa_ref[...], qb_ref[...]], axis=1)
    p1 = -dis * q1
    h2 = _lrelu(
        jnp.dot(h1_ref[...], w20_ref[...],
                preferred_element_type=jnp.float32)
        + jnp.dot(p1, w21_ref[...], preferred_element_type=jnp.float32)
        + b2_ref[...])
    t = jnp.dot(h2, w31_ref[...], preferred_element_type=jnp.float32)
    g2 = dis * t
    g20_ref[...] = g2[:, 0:1]
    g21_ref[...] = g2[:, 1:2]
    h3a_ref[...] = jnp.dot(h2, w30_ref[...],
                           preferred_element_type=jnp.float32) + b3_ref[...]


def _tc2(h1, qa, qb, dis, W20, W21, b2, W30, W31, b3, npad):
    grid = npad // _BLK
    return pl.pallas_call(
        _tc2_body,
        grid=(grid,),
        in_specs=[
            pl.BlockSpec((_BLK, 32), lambda i: (i, 0)),
            pl.BlockSpec((_BLK, 16), lambda i: (i, 0)),
            pl.BlockSpec((_BLK, 16), lambda i: (i, 0)),
            pl.BlockSpec((_BLK, 1), lambda i: (i, 0)),
            pl.BlockSpec((32, 64), lambda i: (0, 0)),
            pl.BlockSpec((32, 64), lambda i: (0, 0)),
            pl.BlockSpec((1, 64), lambda i: (0, 0)),
            pl.BlockSpec((64, 2), lambda i: (0, 0)),
            pl.BlockSpec((64, 2), lambda i: (0, 0)),
            pl.BlockSpec((1, 2), lambda i: (0, 0)),
        ],
        out_specs=[
            pl.BlockSpec((_BLK, 2), lambda i: (i, 0)),
            pl.BlockSpec((_BLK, 1), lambda i: (i, 0)),
            pl.BlockSpec((_BLK, 1), lambda i: (i, 0)),
        ],
        out_shape=[
            jax.ShapeDtypeStruct((npad, 2), jnp.float32),
            jax.ShapeDtypeStruct((npad, 1), jnp.float32),
            jax.ShapeDtypeStruct((npad, 1), jnp.float32),
        ],
    )(h1, qa, qb, dis, W20, W21, b2, W30, W31, b3)


def _tc3_body(nreal, h3a_ref, o00_ref, o01_ref, o10_ref, o11_ref,
              dis_ref, wg_ref, bg_ref, out_ref, acc_ref):
    i = pl.program_id(0)
    ng = pl.num_programs(0)

    @pl.when(i == 0)
    def _():
        acc_ref[0] = -1e30
        acc_ref[1] = 0.0
        acc_ref[2] = 0.0
        acc_ref[3] = 0.0

    q20 = o00_ref[...] + o01_ref[...]
    q21 = o10_ref[...] + o11_ref[...]
    dis = dis_ref[...]
    h3a = h3a_ref[...]
    h30 = h3a[:, 0:1] - dis * q20
    h31 = h3a[:, 1:2] - dis * q21
    wg = wg_ref[...]
    l = h30 * wg[0, 0] + h31 * wg[0, 1] + bg_ref[0, 0]
    rowid = lax.broadcasted_iota(jnp.int32, l.shape, 0) + i * _BLK
    lm = jnp.where(rowid < nreal, l, -1e30)
    m_old = acc_ref[0]
    m_new = jnp.maximum(m_old, jnp.max(lm))
    e = jnp.exp(lm - m_new)
    scale = jnp.exp(m_old - m_new)
    s_new = acc_ref[1] * scale + jnp.sum(e)
    v0 = acc_ref[2] * scale + jnp.sum(e * h30)
    v1 = acc_ref[3] * scale + jnp.sum(e * h31)
    acc_ref[0] = m_new
    acc_ref[1] = s_new
    acc_ref[2] = v0
    acc_ref[3] = v1

    @pl.when(i == ng - 1)
    def _():
        z0 = acc_ref[2] / acc_ref[1]
        z1 = acc_ref[3] / acc_ref[1]
        mz = jnp.maximum(z0, z1)
        lse = mz + jnp.log(jnp.exp(z0 - mz) + jnp.exp(z1 - mz))
        out_ref[...] = jnp.stack([z0 - lse, z1 - lse]).reshape(1, 2)


def _tc3(h3a, o00, o01, o10, o11, dis, Wg, bg, npad, nreal):
    grid = npad // _BLK
    return pl.pallas_call(
        functools.partial(_tc3_body, nreal),
        grid=(grid,),
        in_specs=[
            pl.BlockSpec((_BLK, 2), lambda i: (i, 0)),
            pl.BlockSpec((_BLK, 1), lambda i: (i, 0)),
            pl.BlockSpec((_BLK, 1), lambda i: (i, 0)),
            pl.BlockSpec((_BLK, 1), lambda i: (i, 0)),
            pl.BlockSpec((_BLK, 1), lambda i: (i, 0)),
            pl.BlockSpec((_BLK, 1), lambda i: (i, 0)),
            pl.BlockSpec((1, 2), lambda i: (0, 0)),
            pl.BlockSpec((1, 1), lambda i: (0, 0)),
        ],
        out_specs=pl.BlockSpec((1, 2), lambda i: (0, 0)),
        out_shape=jax.ShapeDtypeStruct((1, 2), jnp.float32),
        scratch_shapes=[pltpu.SMEM((4,), jnp.float32)],
    )(h3a, o00, o01, o10, o11, dis, Wg, bg)


def kernel(x, edge_index, attr, W1, b1, W2, b2, W3, b3, Wg, bg):
    n = x.shape[0]
    e = edge_index.shape[1]
    npad = -(-n // 2048) * 2048
    rows128 = -(-e // 128)
    rows = -(-rows128 // 1024) * 1024
    epad = rows * 128

    src2d = jnp.pad(edge_index[0], (0, epad - e)).reshape(rows, 128)
    dst2d = jnp.pad(edge_index[1], (0, epad - e)).reshape(rows, 128)
    attr2d = jnp.pad(attr, (0, epad - e)).reshape(rows, 128)
    xp = jnp.pad(x, ((0, npad - n), (0, 0)))

    d0, d1 = _make_sc_degree(rows, npad, 8)(src2d, attr2d)
    h1, ga, gb, dis = _tc1(xp, d0.reshape(npad, 1), d1.reshape(npad, 1),
                           W1[0], b1.reshape(1, 32), npad)
    g1f = jnp.concatenate([ga, gb], axis=0)
    qa, qb = _make_sc_q1(rows, npad, 2)(src2d, dst2d, attr2d, g1f)
    h3a, g20, g21 = _tc2(h1, qa, qb, dis, W2[0], W2[1], b2.reshape(1, 64),
                         W3[0], W3[1], b3.reshape(1, 2), npad)
    o00, o01, o10, o11 = _make_sc_q2(rows, npad, 4)(
        src2d, dst2d, attr2d, g20.reshape(npad), g21.reshape(npad))
    out = _tc3(h3a, o00.reshape(npad, 1), o01.reshape(npad, 1),
               o10.reshape(npad, 1), o11.reshape(npad, 1),
               dis, Wg.reshape(1, 2), bg.reshape(1, 1), npad, n)
    return out


# speedup vs baseline: 50.0022x; 1.0002x over previous
import functools

import jax
import jax.numpy as jnp
from jax import lax
from jax.experimental import pallas as pl
from jax.experimental.pallas import tpu as pltpu
from jax.experimental.pallas import tpu_sc as plsc

_LRELU_SLOPE = 0.01
_MB = 8
_NBUF = 4


def _lrelu(x):
    return jnp.where(x >= 0, x, _LRELU_SLOPE * x)


def _zero_vmem_1d(zb_v, n):
    def body(i, _):
        zb_v[pl.ds(i * 16, 16)] = jnp.zeros((16,), jnp.float32)
        return 0
    lax.fori_loop(0, n // 16, body, 0)


def _zero_vmem_rows(zb_v, n):
    def body(i, _):
        zb_v[i, :] = jnp.zeros((16,), jnp.float32)
        return 0
    lax.fori_loop(0, n, body, 0)


def _run_pipeline(nmac, nbuf, ahead, stage, fire, process, drain):
    assert nmac % nbuf == 0 and nmac >= 2 * nbuf and nbuf >= ahead + 2
    for q in range(ahead + 1):
        stage(q, q % nbuf)
    for q in range(ahead):
        fire(q, q % nbuf)

    def emit(m, p, mstat):
        if mstat + ahead < nmac:
            fire(m + ahead, (p + ahead) % nbuf)
        process(m, p)
        if mstat + ahead + 1 < nmac:
            sp = (p + ahead + 1) % nbuf
            if mstat >= nbuf - ahead - 1:
                drain(sp)
            stage(m + ahead + 1, sp)

    for m in range(nbuf):
        emit(m, m % nbuf, m)

    def group(u, _):
        for p in range(nbuf):
            m = nbuf + u * nbuf + p
            emit(m, p, nbuf)
        return 0

    lax.fori_loop(0, (nmac - 2 * nbuf) // nbuf, group, 0)

    for m in range(nmac - nbuf, nmac):
        emit(m, m % nbuf, m)

    for p in range(nbuf):
        drain(p)


def _make_sc_degree(rows, npad, mb):
    rpw = rows // 32
    nmac = rpw // mb
    zb = npad // 128
    stripe = npad // 16
    nb, ahead = 4, 1
    mesh = plsc.VectorSubcoreMesh(core_axis_name="c", subcore_axis_name="s")

    scratch = [pltpu.VMEM_SHARED((npad,), jnp.float32),
               pltpu.VMEM((zb,), jnp.float32)]
    scratch += [pltpu.VMEM((mb, 128), jnp.int32) for _ in range(nb)]
    scratch += [pltpu.VMEM((mb, 128), jnp.float32) for _ in range(nb)]
    scratch += [pltpu.SemaphoreType.DMA for _ in range(2 * nb)]

    @functools.partial(
        pl.kernel,
        mesh=mesh,
        compiler_params=pltpu.CompilerParams(use_tc_tiling_on_sc=False),
        out_type=[jax.ShapeDtypeStruct((npad,), jnp.float32)] * 2,
        scratch_types=scratch,
    )
    def k(src_hbm, attr_hbm, d0_hbm, d1_hbm, deg_sp, zb_v, *bufs):
        srcb = bufs[0:nb]
        attrb = bufs[nb:2 * nb]
        lsem = bufs[2 * nb:3 * nb]
        ssem = bufs[3 * nb:4 * nb]
        c = lax.axis_index("c")
        s = lax.axis_index("s")
        wid = c * 16 + s

        _zero_vmem_1d(zb_v, zb)
        off = s * stripe
        for r in range(8):
            pltpu.sync_copy(zb_v, deg_sp.at[pl.ds(off + r * zb, zb)])
        plsc.subcore_barrier()

        base = wid * rpw

        def stage(m, p):
            r0 = base + m * mb
            pltpu.async_copy(src_hbm.at[pl.ds(r0, mb)], srcb[p], lsem[p])
            pltpu.async_copy(attr_hbm.at[pl.ds(r0, mb)], attrb[p], lsem[p])

        def fire(m, p):
            pass

        def process(m, p):
            pltpu.make_async_copy(src_hbm.at[pl.ds(0, mb)], srcb[p],
                                  lsem[p]).wait()
            pltpu.make_async_copy(attr_hbm.at[pl.ds(0, mb)], attrb[p],
                                  lsem[p]).wait()
            for j in range(mb):
                pltpu.async_copy(attrb[p].at[j], deg_sp.at[srcb[p].at[j]],
                                 ssem[p], add=True)

        def drain(p):
            for j in range(mb):
                pltpu.make_async_copy(attrb[p].at[0],
                                      deg_sp.at[srcb[p].at[0]],
                                      ssem[p]).wait()

        _run_pipeline(nmac, nb, ahead, stage, fire, process, drain)
        plsc.subcore_barrier()

        @pl.when(c == 0)
        def _():
            pltpu.sync_copy(deg_sp.at[pl.ds(off, stripe)],
                            d0_hbm.at[pl.ds(off, stripe)])

        @pl.when(c == 1)
        def _():
            pltpu.sync_copy(deg_sp.at[pl.ds(off, stripe)],
                            d1_hbm.at[pl.ds(off, stripe)])

    return k


def _make_sc_q1(rows, npad, mb):
    rpw = rows // 16
    nmac = rpw // mb
    zb = npad // 128
    stripe = npad // 16
    nb, ahead = 5, 2
    mesh = plsc.VectorSubcoreMesh(core_axis_name="c", subcore_axis_name="s")

    scratch = [pltpu.VMEM_SHARED((npad, 16), jnp.float32),
               pltpu.VMEM((128, 16), jnp.float32)]
    scratch += [pltpu.VMEM((mb, 128), jnp.int32) for _ in range(nb)]
    scratch += [pltpu.VMEM((mb, 128), jnp.int32) for _ in range(nb)]
    scratch += [pltpu.VMEM((mb, 128), jnp.float32) for _ in range(nb)]
    scratch += [pltpu.VMEM((mb * 128, 16), jnp.float32)
                for _ in range(nb)]
    scratch += [pltpu.SemaphoreType.DMA for _ in range(3 * nb)]

    @functools.partial(
        pl.kernel,
        mesh=mesh,
        compiler_params=pltpu.CompilerParams(use_tc_tiling_on_sc=False),
        out_type=[jax.ShapeDtypeStruct((npad, 16), jnp.float32)] * 2,
        scratch_types=scratch,
    )
    def k(src_hbm, dst_hbm, attr_hbm, ga_hbm, gb_hbm, qa_hbm, qb_hbm,
          q_sp, zb_v, *bufs):
        srcb = bufs[0:nb]
        dstb = bufs[nb:2 * nb]
        attrb = bufs[2 * nb:3 * nb]
        rowsb = bufs[3 * nb:4 * nb]
        lsem = bufs[4 * nb:5 * nb]
        gsem = bufs[5 * nb:6 * nb]
        ssem = bufs[6 * nb:7 * nb]
        c = lax.axis_index("c")
        s = lax.axis_index("s")

        _zero_vmem_rows(zb_v, 128)
        off = s * stripe
        for r in range(stripe // 128):
            pltpu.sync_copy(zb_v, q_sp.at[pl.ds(off + r * 128, 128)])
        plsc.subcore_barrier()

        base = s * rpw

        def stage(m, p):
            r0 = base + m * mb
            pltpu.async_copy(src_hbm.at[pl.ds(r0, mb)], srcb[p], lsem[p])
            pltpu.async_copy(dst_hbm.at[pl.ds(r0, mb)], dstb[p], lsem[p])
            pltpu.async_copy(attr_hbm.at[pl.ds(r0, mb)], attrb[p], lsem[p])

        def fire(m, p):
            for _ in range(3):
                pltpu.make_async_copy(attr_hbm.at[pl.ds(0, mb)], attrb[p],
                                      lsem[p]).wait()

            @pl.when(c == 0)
            def _():
                for j in range(mb):
                    pltpu.async_copy(ga_hbm.at[srcb[p].at[j]],
                                     rowsb[p].at[pl.ds(j * 128, 128)],
                                     gsem[p])

            @pl.when(c == 1)
            def _():
                for j in range(mb):
                    pltpu.async_copy(gb_hbm.at[srcb[p].at[j]],
                                     rowsb[p].at[pl.ds(j * 128, 128)],
                                     gsem[p])

        def process(m, p):
            for _ in range(mb):
                pltpu.make_async_copy(ga_hbm.at[srcb[p].at[0]],
                                      rowsb[p].at[pl.ds(0, 128)],
                                      gsem[p]).wait()

            def gbody(g, _):
                j = g >> 3
                av = attrb[p][j, pl.ds((g & 7) * 16, 16)]
                for li in range(16):
                    i = g * 16 + li
                    rowsb[p][i, :] = rowsb[p][i, :] * av[li]
                return 0

            lax.fori_loop(0, mb * 8, gbody, 0)
            for j in range(mb):
                pltpu.async_copy(rowsb[p].at[pl.ds(j * 128, 128)],
                                 q_sp.at[dstb[p].at[j]], ssem[p], add=True)

        def drain(p):
            for _ in range(mb):
                pltpu.make_async_copy(rowsb[p].at[pl.ds(0, 128)],
                                      q_sp.at[dstb[p].at[0]],
                                      ssem[p]).wait()

        _run_pipeline(nmac, nb, ahead, stage, fire, process, drain)
        plsc.subcore_barrier()

        @pl.when(c == 0)
        def _():
            pltpu.sync_copy(q_sp.at[pl.ds(off, stripe)],
                            qa_hbm.at[pl.ds(off, stripe)])

        @pl.when(c == 1)
        def _():
            pltpu.sync_copy(q_sp.at[pl.ds(off, stripe)],
                            qb_hbm.at[pl.ds(off, stripe)])

    return k


def _make_sc_q2(rows, npad, mb):
    rpw = rows // 32
    nmac = rpw // mb
    zb = npad // 128
    stripe = npad // 16
    nb, ahead = 5, 2
    mesh = plsc.VectorSubcoreMesh(core_axis_name="c", subcore_axis_name="s")

    scratch = [pltpu.VMEM_SHARED((npad,), jnp.float32) for _ in range(4)]
    scratch += [pltpu.VMEM((zb,), jnp.float32)]
    scratch += [pltpu.VMEM((mb, 128), jnp.int32) for _ in range(nb)]
    scratch += [pltpu.VMEM((mb, 128), jnp.int32) for _ in range(nb)]
    scratch += [pltpu.VMEM((mb, 128), jnp.float32) for _ in range(nb)]
    scratch += [pltpu.VMEM((mb * 128,), jnp.float32) for _ in range(nb)]
    scratch += [pltpu.VMEM((mb * 128,), jnp.float32) for _ in range(nb)]
    scratch += [pltpu.SemaphoreType.DMA for _ in range(3 * nb)]

    @functools.partial(
        pl.kernel,
        mesh=mesh,
        compiler_params=pltpu.CompilerParams(use_tc_tiling_on_sc=False),
        out_type=[jax.ShapeDtypeStruct((npad,), jnp.float32)] * 4,
        scratch_types=scratch,
    )
    def k(src_hbm, dst_hbm, attr_hbm, g20_hbm, g21_hbm,
          o00_hbm, o01_hbm, o10_hbm, o11_hbm,
          g0_sp, g1_sp, q0_sp, q1_sp, zb_v, *bufs):
        srcb = bufs[0:nb]
        dstb = bufs[nb:2 * nb]
        attrb = bufs[2 * nb:3 * nb]
        r0b = bufs[3 * nb:4 * nb]
        r1b = bufs[4 * nb:5 * nb]
        lsem = bufs[5 * nb:6 * nb]
        gsem = bufs[6 * nb:7 * nb]
        ssem = bufs[7 * nb:8 * nb]
        c = lax.axis_index("c")
        s = lax.axis_index("s")
        wid = c * 16 + s

        off = s * stripe
        pltpu.sync_copy(g20_hbm.at[pl.ds(off, stripe)],
                        g0_sp.at[pl.ds(off, stripe)])
        pltpu.sync_copy(g21_hbm.at[pl.ds(off, stripe)],
                        g1_sp.at[pl.ds(off, stripe)])
        _zero_vmem_1d(zb_v, zb)
        for r in range(8):
            pltpu.sync_copy(zb_v, q0_sp.at[pl.ds(off + r * zb, zb)])
            pltpu.sync_copy(zb_v, q1_sp.at[pl.ds(off + r * zb, zb)])
        plsc.subcore_barrier()

        base = wid * rpw

        def stage(m, p):
            r0 = base + m * mb
            pltpu.async_copy(src_hbm.at[pl.ds(r0, mb)], srcb[p], lsem[p])
            pltpu.async_copy(dst_hbm.at[pl.ds(r0, mb)], dstb[p], lsem[p])
            pltpu.async_copy(attr_hbm.at[pl.ds(r0, mb)], attrb[p], lsem[p])

        def fire(m, p):
            for _ in range(3):
                pltpu.make_async_copy(attr_hbm.at[pl.ds(0, mb)], attrb[p],
                                      lsem[p]).wait()
            for j in range(mb):
                sl = pl.ds(j * 128, 128)
                pltpu.async_copy(g0_sp.at[srcb[p].at[j]], r0b[p].at[sl],
                                 gsem[p])
                pltpu.async_copy(g1_sp.at[srcb[p].at[j]], r1b[p].at[sl],
                                 gsem[p])

        def process(m, p):
            for _ in range(2 * mb):
                pltpu.make_async_copy(g0_sp.at[srcb[p].at[0]],
                                      r0b[p].at[pl.ds(0, 128)],
                                      gsem[p]).wait()

            def gbody(g, _):
                j = g >> 3
                sl16 = pl.ds(g * 16, 16)
                av = attrb[p][j, pl.ds((g & 7) * 16, 16)]
                r0b[p][sl16] = r0b[p][sl16] * av
                r1b[p][sl16] = r1b[p][sl16] * av
                return 0

            lax.fori_loop(0, mb * 8, gbody, 0)
            for j in range(mb):
                sl = pl.ds(j * 128, 128)
                pltpu.async_copy(r0b[p].at[sl], q0_sp.at[dstb[p].at[j]],
                                 ssem[p], add=True)
                pltpu.async_copy(r1b[p].at[sl], q1_sp.at[dstb[p].at[j]],
                                 ssem[p], add=True)

        def drain(p):
            for _ in range(2 * mb):
                pltpu.make_async_copy(r0b[p].at[pl.ds(0, 128)],
                                      q0_sp.at[dstb[p].at[0]],
                                      ssem[p]).wait()

        _run_pipeline(nmac, nb, ahead, stage, fire, process, drain)
        plsc.subcore_barrier()

        @pl.when(c == 0)
        def _():
            pltpu.sync_copy(q0_sp.at[pl.ds(off, stripe)],
                            o00_hbm.at[pl.ds(off, stripe)])
            pltpu.sync_copy(q1_sp.at[pl.ds(off, stripe)],
                            o10_hbm.at[pl.ds(off, stripe)])

        @pl.when(c == 1)
        def _():
            pltpu.sync_copy(q0_sp.at[pl.ds(off, stripe)],
                            o01_hbm.at[pl.ds(off, stripe)])
            pltpu.sync_copy(q1_sp.at[pl.ds(off, stripe)],
                            o11_hbm.at[pl.ds(off, stripe)])

    return k


_BLK = 2048


def _tc1_body(x_ref, d0_ref, d1_ref, w_ref, b_ref,
              h1_ref, ga_ref, gb_ref, dis_ref):
    deg = d0_ref[...] + d1_ref[...]
    dis = jnp.where(deg > 0, lax.rsqrt(deg), 0.0)
    h = jnp.dot(x_ref[...], w_ref[...],
                preferred_element_type=jnp.float32) + b_ref[...]
    h1 = _lrelu(h)
    h1_ref[...] = h1
    g = dis * h1
    ga_ref[...] = g[:, :16]
    gb_ref[...] = g[:, 16:]
    dis_ref[...] = dis


def _tc1(xp, d0, d1, W10, b1, npad):
    grid = npad // _BLK
    return pl.pallas_call(
        _tc1_body,
        grid=(grid,),
        in_specs=[
            pl.BlockSpec((_BLK, 20), lambda i: (i, 0)),
            pl.BlockSpec((_BLK, 1), lambda i: (i, 0)),
            pl.BlockSpec((_BLK, 1), lambda i: (i, 0)),
            pl.BlockSpec((20, 32), lambda i: (0, 0)),
            pl.BlockSpec((1, 32), lambda i: (0, 0)),
        ],
        out_specs=[
            pl.BlockSpec((_BLK, 32), lambda i: (i, 0)),
            pl.BlockSpec((_BLK, 16), lambda i: (i, 0)),
            pl.BlockSpec((_BLK, 16), lambda i: (i, 0)),
            pl.BlockSpec((_BLK, 1), lambda i: (i, 0)),
        ],
        out_shape=[
            jax.ShapeDtypeStruct((npad, 32), jnp.float32),
            jax.ShapeDtypeStruct((npad, 16), jnp.float32),
            jax.ShapeDtypeStruct((npad, 16), jnp.float32),
            jax.ShapeDtypeStruct((npad, 1), jnp.float32),
        ],
    )(xp, d0, d1, W10, b1)


def _tc2_body(h1_ref, qa_ref, qb_ref, dis_ref,
              w20_ref, w21_ref, b2_ref, w30_ref, w31_ref, b3_ref,
              h3a_ref, g20_ref, g21_ref):
    dis = dis_ref[...]
    q1 = jnp.concatenate([qa_ref[...], qb_ref[...]], axis=1)
    p1 = -dis * q1
    h2 = _lrelu(
        jnp.dot(h1_ref[...], w20_ref[...],
                preferred_element_type=jnp.float32)
        + jnp.dot(p1, w21_ref[...], preferred_element_type=jnp.float32)
        + b2_ref[...])
    t = jnp.dot(h2, w31_ref[...], preferred_element_type=jnp.float32)
    g2 = dis * t
    g20_ref[...] = g2[:, 0:1]
    g21_ref[...] = g2[:, 1:2]
    h3a_ref[...] = jnp.dot(h2, w30_ref[...],
                           preferred_element_type=jnp.float32) + b3_ref[...]


def _tc2(h1, qa, qb, dis, W20, W21, b2, W30, W31, b3, npad):
    grid = npad // _BLK
    return pl.pallas_call(
        _tc2_body,
        grid=(grid,),
        in_specs=[
            pl.BlockSpec((_BLK, 32), lambda i: (i, 0)),
            pl.BlockSpec((_BLK, 16), lambda i: (i, 0)),
            pl.BlockSpec((_BLK, 16), lambda i: (i, 0)),
            pl.BlockSpec((_BLK, 1), lambda i: (i, 0)),
            pl.BlockSpec((32, 64), lambda i: (0, 0)),
            pl.BlockSpec((32, 64), lambda i: (0, 0)),
            pl.BlockSpec((1, 64), lambda i: (0, 0)),
            pl.BlockSpec((64, 2), lambda i: (0, 0)),
            pl.BlockSpec((64, 2), lambda i: (0, 0)),
            pl.BlockSpec((1, 2), lambda i: (0, 0)),
        ],
        out_specs=[
            pl.BlockSpec((_BLK, 2), lambda i: (i, 0)),
            pl.BlockSpec((_BLK, 1), lambda i: (i, 0)),
            pl.BlockSpec((_BLK, 1), lambda i: (i, 0)),
        ],
        out_shape=[
            jax.ShapeDtypeStruct((npad, 2), jnp.float32),
            jax.ShapeDtypeStruct((npad, 1), jnp.float32),
            jax.ShapeDtypeStruct((npad, 1), jnp.float32),
        ],
    )(h1, qa, qb, dis, W20, W21, b2, W30, W31, b3)


def _tc3_body(nreal, h3a_ref, o00_ref, o01_ref, o10_ref, o11_ref,
              dis_ref, wg_ref, bg_ref, out_ref, acc_ref):
    i = pl.program_id(0)
    ng = pl.num_programs(0)

    @pl.when(i == 0)
    def _():
        acc_ref[0] = -1e30
        acc_ref[1] = 0.0
        acc_ref[2] = 0.0
        acc_ref[3] = 0.0

    q20 = o00_ref[...] + o01_ref[...]
    q21 = o10_ref[...] + o11_ref[...]
    dis = dis_ref[...]
    h3a = h3a_ref[...]
    h30 = h3a[:, 0:1] - dis * q20
    h31 = h3a[:, 1:2] - dis * q21
    wg = wg_ref[...]
    l = h30 * wg[0, 0] + h31 * wg[0, 1] + bg_ref[0, 0]
    rowid = lax.broadcasted_iota(jnp.int32, l.shape, 0) + i * _BLK
    lm = jnp.where(rowid < nreal, l, -1e30)
    m_old = acc_ref[0]
    m_new = jnp.maximum(m_old, jnp.max(lm))
    e = jnp.exp(lm - m_new)
    scale = jnp.exp(m_old - m_new)
    s_new = acc_ref[1] * scale + jnp.sum(e)
    v0 = acc_ref[2] * scale + jnp.sum(e * h30)
    v1 = acc_ref[3] * scale + jnp.sum(e * h31)
    acc_ref[0] = m_new
    acc_ref[1] = s_new
    acc_ref[2] = v0
    acc_ref[3] = v1

    @pl.when(i == ng - 1)
    def _():
        z0 = acc_ref[2] / acc_ref[1]
        z1 = acc_ref[3] / acc_ref[1]
        mz = jnp.maximum(z0, z1)
        lse = mz + jnp.log(jnp.exp(z0 - mz) + jnp.exp(z1 - mz))
        out_ref[...] = jnp.stack([z0 - lse, z1 - lse]).reshape(1, 2)


def _tc3(h3a, o00, o01, o10, o11, dis, Wg, bg, npad, nreal):
    grid = npad // _BLK
    return pl.pallas_call(
        functools.partial(_tc3_body, nreal),
        grid=(grid,),
        in_specs=[
            pl.BlockSpec((_BLK, 2), lambda i: (i, 0)),
            pl.BlockSpec((_BLK, 1), lambda i: (i, 0)),
            pl.BlockSpec((_BLK, 1), lambda i: (i, 0)),
            pl.BlockSpec((_BLK, 1), lambda i: (i, 0)),
            pl.BlockSpec((_BLK, 1), lambda i: (i, 0)),
            pl.BlockSpec((_BLK, 1), lambda i: (i, 0)),
            pl.BlockSpec((1, 2), lambda i: (0, 0)),
            pl.BlockSpec((1, 1), lambda i: (0, 0)),
        ],
        out_specs=pl.BlockSpec((1, 2), lambda i: (0, 0)),
        out_shape=jax.ShapeDtypeStruct((1, 2), jnp.float32),
        scratch_shapes=[pltpu.SMEM((4,), jnp.float32)],
    )(h3a, o00, o01, o10, o11, dis, Wg, bg)


def _prep_body(e, rows_out, src_r, dst_r, attr_r, src_o, dst_o, attr_o):
    i = pl.program_id(0)
    pos = (i * _PR * 128
           + lax.broadcasted_iota(jnp.int32, (_PR, 128), 0) * 128
           + lax.broadcasted_iota(jnp.int32, (_PR, 128), 1))
    valid = pos < e
    src_o[...] = jnp.where(valid, src_r[...], 0)
    dst_o[...] = jnp.where(valid, dst_r[...], 0)
    attr_o[...] = jnp.where(valid, attr_r[...], 0.0)


_PR = 200


def _prep(src_r, dst_r, attr_r, rows_in, rows_out, e):
    grid = rows_out // _PR
    lim = rows_in // _PR - 1
    spec_in = pl.BlockSpec((_PR, 128), lambda i: (jnp.minimum(i, lim), 0))
    spec_out = pl.BlockSpec((_PR, 128), lambda i: (i, 0))
    return pl.pallas_call(
        functools.partial(_prep_body, e, rows_out),
        grid=(grid,),
        in_specs=[spec_in] * 3,
        out_specs=[spec_out] * 3,
        out_shape=[
            jax.ShapeDtypeStruct((rows_out, 128), jnp.int32),
            jax.ShapeDtypeStruct((rows_out, 128), jnp.int32),
            jax.ShapeDtypeStruct((rows_out, 128), jnp.float32),
        ],
    )(src_r, dst_r, attr_r)


def kernel(x, edge_index, attr, W1, b1, W2, b2, W3, b3, Wg, bg):
    n = x.shape[0]
    e = edge_index.shape[1]
    npad = -(-n // 2048) * 2048
    rows128 = -(-e // 128)
    rows = -(-rows128 // 1024) * 1024
    epad = rows * 128

    if e % (128 * _PR) == 0:
        src2d, dst2d, attr2d = _prep(
            edge_index[0].reshape(rows128, 128),
            edge_index[1].reshape(rows128, 128),
            attr.reshape(rows128, 128), rows128, rows, e)
    else:
        src2d = jnp.pad(edge_index[0], (0, epad - e)).reshape(rows, 128)
        dst2d = jnp.pad(edge_index[1], (0, epad - e)).reshape(rows, 128)
        attr2d = jnp.pad(attr, (0, epad - e)).reshape(rows, 128)

    d0, d1 = _make_sc_degree(rows, npad, 8)(src2d, attr2d)
    h1, ga, gb, dis = _tc1(x, d0.reshape(npad, 1), d1.reshape(npad, 1),
                           W1[0], b1.reshape(1, 32), npad)
    qa, qb = _make_sc_q1(rows, npad, 2)(src2d, dst2d, attr2d, ga, gb)
    h3a, g20, g21 = _tc2(h1, qa, qb, dis, W2[0], W2[1], b2.reshape(1, 64),
                         W3[0], W3[1], b3.reshape(1, 2), npad)
    o00, o01, o10, o11 = _make_sc_q2(rows, npad, 4)(
        src2d, dst2d, attr2d, g20.reshape(npad), g21.reshape(npad))
    out = _tc3(h3a, o00.reshape(npad, 1), o01.reshape(npad, 1),
               o10.reshape(npad, 1), o11.reshape(npad, 1),
               dis, Wg.reshape(1, 2), bg.reshape(1, 1), npad, n)
    return out


# speedup vs baseline: 50.4366x; 1.0087x over previous
import functools

import jax
import jax.numpy as jnp
from jax import lax
from jax.experimental import pallas as pl
from jax.experimental.pallas import tpu as pltpu
from jax.experimental.pallas import tpu_sc as plsc

_LRELU_SLOPE = 0.01
_MB = 8
_NBUF = 4


def _lrelu(x):
    return jnp.where(x >= 0, x, _LRELU_SLOPE * x)


def _zero_vmem_1d(zb_v, n):
    def body(i, _):
        zb_v[pl.ds(i * 16, 16)] = jnp.zeros((16,), jnp.float32)
        return 0
    lax.fori_loop(0, n // 16, body, 0)


def _zero_vmem_rows(zb_v, n):
    def body(i, _):
        zb_v[i, :] = jnp.zeros((16,), jnp.float32)
        return 0
    lax.fori_loop(0, n, body, 0)


def _run_pipeline(nmac, nbuf, ahead, stage, fire, process, drain):
    assert nmac % nbuf == 0 and nmac >= 2 * nbuf and nbuf >= ahead + 2
    for q in range(ahead + 1):
        stage(q, q % nbuf)
    for q in range(ahead):
        fire(q, q % nbuf)

    def emit(m, p, mstat):
        if mstat + ahead < nmac:
            fire(m + ahead, (p + ahead) % nbuf)
        process(m, p)
        if mstat + ahead + 1 < nmac:
            sp = (p + ahead + 1) % nbuf
            if mstat >= nbuf - ahead - 1:
                drain(sp)
            stage(m + ahead + 1, sp)

    for m in range(nbuf):
        emit(m, m % nbuf, m)

    def group(u, _):
        for p in range(nbuf):
            m = nbuf + u * nbuf + p
            emit(m, p, nbuf)
        return 0

    lax.fori_loop(0, (nmac - 2 * nbuf) // nbuf, group, 0)

    for m in range(nmac - nbuf, nmac):
        emit(m, m % nbuf, m)

    for p in range(nbuf):
        drain(p)


def _make_sc_degree(rows, npad, mb):
    rpw = rows // 32
    nmac = rpw // mb
    zb = npad // 128
    stripe = npad // 16
    nb, ahead = 4, 1
    mesh = plsc.VectorSubcoreMesh(core_axis_name="c", subcore_axis_name="s")

    scratch = [pltpu.VMEM_SHARED((npad,), jnp.float32),
               pltpu.VMEM((zb,), jnp.float32)]
    scratch += [pltpu.VMEM((mb, 128), jnp.int32) for _ in range(nb)]
    scratch += [pltpu.VMEM((mb, 128), jnp.float32) for _ in range(nb)]
    scratch += [pltpu.SemaphoreType.DMA for _ in range(2 * nb)]

    @functools.partial(
        pl.kernel,
        mesh=mesh,
        compiler_params=pltpu.CompilerParams(use_tc_tiling_on_sc=False),
        out_type=[jax.ShapeDtypeStruct((npad,), jnp.float32)] * 2,
        scratch_types=scratch,
    )
    def k(src_hbm, attr_hbm, d0_hbm, d1_hbm, deg_sp, zb_v, *bufs):
        srcb = bufs[0:nb]
        attrb = bufs[nb:2 * nb]
        lsem = bufs[2 * nb:3 * nb]
        ssem = bufs[3 * nb:4 * nb]
        c = lax.axis_index("c")
        s = lax.axis_index("s")
        wid = c * 16 + s

        _zero_vmem_1d(zb_v, zb)
        off = s * stripe
        for r in range(8):
            pltpu.sync_copy(zb_v, deg_sp.at[pl.ds(off + r * zb, zb)])
        plsc.subcore_barrier()

        base = wid * rpw

        def stage(m, p):
            r0 = base + m * mb
            pltpu.async_copy(src_hbm.at[pl.ds(r0, mb)], srcb[p], lsem[p])
            pltpu.async_copy(attr_hbm.at[pl.ds(r0, mb)], attrb[p], lsem[p])

        def fire(m, p):
            pass

        def process(m, p):
            pltpu.make_async_copy(src_hbm.at[pl.ds(0, mb)], srcb[p],
                                  lsem[p]).wait()
            pltpu.make_async_copy(attr_hbm.at[pl.ds(0, mb)], attrb[p],
                                  lsem[p]).wait()
            for j in range(mb):
                pltpu.async_copy(attrb[p].at[j], deg_sp.at[srcb[p].at[j]],
                                 ssem[p], add=True)

        def drain(p):
            for j in range(mb):
                pltpu.make_async_copy(attrb[p].at[0],
                                      deg_sp.at[srcb[p].at[0]],
                                      ssem[p]).wait()

        _run_pipeline(nmac, nb, ahead, stage, fire, process, drain)
        plsc.subcore_barrier()

        @pl.when(c == 0)
        def _():
            pltpu.sync_copy(deg_sp.at[pl.ds(off, stripe)],
                            d0_hbm.at[pl.ds(off, stripe)])

        @pl.when(c == 1)
        def _():
            pltpu.sync_copy(deg_sp.at[pl.ds(off, stripe)],
                            d1_hbm.at[pl.ds(off, stripe)])

    return k


def _make_sc_q1(rows, npad, mb):
    rpw = rows // 16
    nmac = rpw // mb
    zb = npad // 128
    stripe = npad // 16
    nb, ahead = 5, 2
    mesh = plsc.VectorSubcoreMesh(core_axis_name="c", subcore_axis_name="s")

    scratch = [pltpu.VMEM_SHARED((npad, 16), jnp.float32),
               pltpu.VMEM((128, 16), jnp.float32)]
    scratch += [pltpu.VMEM((mb, 128), jnp.int32) for _ in range(nb)]
    scratch += [pltpu.VMEM((mb, 128), jnp.int32) for _ in range(nb)]
    scratch += [pltpu.VMEM((mb, 128), jnp.float32) for _ in range(nb)]
    scratch += [pltpu.VMEM((mb * 128, 16), jnp.float32)
                for _ in range(nb)]
    scratch += [pltpu.SemaphoreType.DMA for _ in range(3 * nb)]

    @functools.partial(
        pl.kernel,
        mesh=mesh,
        compiler_params=pltpu.CompilerParams(use_tc_tiling_on_sc=False),
        out_type=[jax.ShapeDtypeStruct((npad, 16), jnp.float32)] * 2,
        scratch_types=scratch,
    )
    def k(src_hbm, dst_hbm, attr_hbm, ga_hbm, gb_hbm, qa_hbm, qb_hbm,
          q_sp, zb_v, *bufs):
        srcb = bufs[0:nb]
        dstb = bufs[nb:2 * nb]
        attrb = bufs[2 * nb:3 * nb]
        rowsb = bufs[3 * nb:4 * nb]
        lsem = bufs[4 * nb:5 * nb]
        gsem = bufs[5 * nb:6 * nb]
        ssem = bufs[6 * nb:7 * nb]
        c = lax.axis_index("c")
        s = lax.axis_index("s")

        _zero_vmem_rows(zb_v, 128)
        off = s * stripe
        for r in range(stripe // 128):
            pltpu.sync_copy(zb_v, q_sp.at[pl.ds(off + r * 128, 128)])
        plsc.subcore_barrier()

        base = s * rpw

        def stage(m, p):
            r0 = base + m * mb
            pltpu.async_copy(src_hbm.at[pl.ds(r0, mb)], srcb[p], lsem[p])
            pltpu.async_copy(dst_hbm.at[pl.ds(r0, mb)], dstb[p], lsem[p])
            pltpu.async_copy(attr_hbm.at[pl.ds(r0, mb)], attrb[p], lsem[p])

        def fire(m, p):
            for _ in range(3):
                pltpu.make_async_copy(attr_hbm.at[pl.ds(0, mb)], attrb[p],
                                      lsem[p]).wait()

            @pl.when(c == 0)
            def _():
                for j in range(mb):
                    pltpu.async_copy(ga_hbm.at[srcb[p].at[j]],
                                     rowsb[p].at[pl.ds(j * 128, 128)],
                                     gsem[p])

            @pl.when(c == 1)
            def _():
                for j in range(mb):
                    pltpu.async_copy(gb_hbm.at[srcb[p].at[j]],
                                     rowsb[p].at[pl.ds(j * 128, 128)],
                                     gsem[p])

        def process(m, p):
            for _ in range(mb):
                pltpu.make_async_copy(ga_hbm.at[srcb[p].at[0]],
                                      rowsb[p].at[pl.ds(0, 128)],
                                      gsem[p]).wait()

            def gbody(g, _):
                j = g >> 3
                av = attrb[p][j, pl.ds((g & 7) * 16, 16)]
                for li in range(16):
                    i = g * 16 + li
                    rowsb[p][i, :] = rowsb[p][i, :] * av[li]
                return 0

            lax.fori_loop(0, mb * 8, gbody, 0)
            for j in range(mb):
                pltpu.async_copy(rowsb[p].at[pl.ds(j * 128, 128)],
                                 q_sp.at[dstb[p].at[j]], ssem[p], add=True)

        def drain(p):
            for _ in range(mb):
                pltpu.make_async_copy(rowsb[p].at[pl.ds(0, 128)],
                                      q_sp.at[dstb[p].at[0]],
                                      ssem[p]).wait()

        _run_pipeline(nmac, nb, ahead, stage, fire, process, drain)
        plsc.subcore_barrier()

        @pl.when(c == 0)
        def _():
            pltpu.sync_copy(q_sp.at[pl.ds(off, stripe)],
                            qa_hbm.at[pl.ds(off, stripe)])

        @pl.when(c == 1)
        def _():
            pltpu.sync_copy(q_sp.at[pl.ds(off, stripe)],
                            qb_hbm.at[pl.ds(off, stripe)])

    return k


def _make_sc_q2(rows, npad, mb):
    rpw = rows // 32
    nmac = rpw // mb
    zb = npad // 128
    stripe = npad // 16
    nb, ahead = 5, 2
    mesh = plsc.VectorSubcoreMesh(core_axis_name="c", subcore_axis_name="s")

    scratch = [pltpu.VMEM_SHARED((npad,), jnp.float32) for _ in range(4)]
    scratch += [pltpu.VMEM((zb,), jnp.float32)]
    scratch += [pltpu.VMEM((mb, 128), jnp.int32) for _ in range(nb)]
    scratch += [pltpu.VMEM((mb, 128), jnp.int32) for _ in range(nb)]
    scratch += [pltpu.VMEM((mb, 128), jnp.float32) for _ in range(nb)]
    scratch += [pltpu.VMEM((mb * 128,), jnp.float32) for _ in range(nb)]
    scratch += [pltpu.VMEM((mb * 128,), jnp.float32) for _ in range(nb)]
    scratch += [pltpu.SemaphoreType.DMA for _ in range(3 * nb)]

    @functools.partial(
        pl.kernel,
        mesh=mesh,
        compiler_params=pltpu.CompilerParams(use_tc_tiling_on_sc=False),
        out_type=[jax.ShapeDtypeStruct((npad,), jnp.float32)] * 4,
        scratch_types=scratch,
    )
    def k(src_hbm, dst_hbm, attr_hbm, g20_hbm, g21_hbm,
          o00_hbm, o01_hbm, o10_hbm, o11_hbm,
          g0_sp, g1_sp, q0_sp, q1_sp, zb_v, *bufs):
        srcb = bufs[0:nb]
        dstb = bufs[nb:2 * nb]
        attrb = bufs[2 * nb:3 * nb]
        r0b = bufs[3 * nb:4 * nb]
        r1b = bufs[4 * nb:5 * nb]
        lsem = bufs[5 * nb:6 * nb]
        gsem = bufs[6 * nb:7 * nb]
        ssem = bufs[7 * nb:8 * nb]
        c = lax.axis_index("c")
        s = lax.axis_index("s")
        wid = c * 16 + s

        off = s * stripe
        pltpu.sync_copy(g20_hbm.at[pl.ds(off, stripe)],
                        g0_sp.at[pl.ds(off, stripe)])
        pltpu.sync_copy(g21_hbm.at[pl.ds(off, stripe)],
                        g1_sp.at[pl.ds(off, stripe)])
        _zero_vmem_1d(zb_v, zb)
        for r in range(8):
            pltpu.sync_copy(zb_v, q0_sp.at[pl.ds(off + r * zb, zb)])
            pltpu.sync_copy(zb_v, q1_sp.at[pl.ds(off + r * zb, zb)])
        plsc.subcore_barrier()

        base = wid * rpw

        def stage(m, p):
            r0 = base + m * mb
            pltpu.async_copy(src_hbm.at[pl.ds(r0, mb)], srcb[p], lsem[p])
            pltpu.async_copy(dst_hbm.at[pl.ds(r0, mb)], dstb[p], lsem[p])
            pltpu.async_copy(attr_hbm.at[pl.ds(r0, mb)], attrb[p], lsem[p])

        def fire(m, p):
            for _ in range(3):
                pltpu.make_async_copy(attr_hbm.at[pl.ds(0, mb)], attrb[p],
                                      lsem[p]).wait()
            for j in range(mb):
                sl = pl.ds(j * 128, 128)
                pltpu.async_copy(g0_sp.at[srcb[p].at[j]], r0b[p].at[sl],
                                 gsem[p])
                pltpu.async_copy(g1_sp.at[srcb[p].at[j]], r1b[p].at[sl],
                                 gsem[p])

        def process(m, p):
            for _ in range(2 * mb):
                pltpu.make_async_copy(g0_sp.at[srcb[p].at[0]],
                                      r0b[p].at[pl.ds(0, 128)],
                                      gsem[p]).wait()

            def gbody(g, _):
                j = g >> 3
                sl16 = pl.ds(g * 16, 16)
                av = attrb[p][j, pl.ds((g & 7) * 16, 16)]
                r0b[p][sl16] = r0b[p][sl16] * av
                r1b[p][sl16] = r1b[p][sl16] * av
                return 0

            lax.fori_loop(0, mb * 8, gbody, 0)
            for j in range(mb):
                sl = pl.ds(j * 128, 128)
                pltpu.async_copy(r0b[p].at[sl], q0_sp.at[dstb[p].at[j]],
                                 ssem[p], add=True)
                pltpu.async_copy(r1b[p].at[sl], q1_sp.at[dstb[p].at[j]],
                                 ssem[p], add=True)

        def drain(p):
            for _ in range(2 * mb):
                pltpu.make_async_copy(r0b[p].at[pl.ds(0, 128)],
                                      q0_sp.at[dstb[p].at[0]],
                                      ssem[p]).wait()

        _run_pipeline(nmac, nb, ahead, stage, fire, process, drain)
        plsc.subcore_barrier()

        @pl.when(c == 0)
        def _():
            pltpu.sync_copy(q0_sp.at[pl.ds(off, stripe)],
                            o00_hbm.at[pl.ds(off, stripe)])
            pltpu.sync_copy(q1_sp.at[pl.ds(off, stripe)],
                            o10_hbm.at[pl.ds(off, stripe)])

        @pl.when(c == 1)
        def _():
            pltpu.sync_copy(q0_sp.at[pl.ds(off, stripe)],
                            o01_hbm.at[pl.ds(off, stripe)])
            pltpu.sync_copy(q1_sp.at[pl.ds(off, stripe)],
                            o11_hbm.at[pl.ds(off, stripe)])

    return k


_BLK = 2048


def _tc1a_body(x_ref, w_ref, b_ref, h1_ref):
    h = jnp.dot(x_ref[...], w_ref[...],
                preferred_element_type=jnp.float32) + b_ref[...]
    h1_ref[...] = _lrelu(h)


def _tc1a(xp, W10, b1, npad):
    grid = npad // _BLK
    return pl.pallas_call(
        _tc1a_body,
        grid=(grid,),
        in_specs=[
            pl.BlockSpec((_BLK, 20), lambda i: (i, 0)),
            pl.BlockSpec((20, 32), lambda i: (0, 0)),
            pl.BlockSpec((1, 32), lambda i: (0, 0)),
        ],
        out_specs=pl.BlockSpec((_BLK, 32), lambda i: (i, 0)),
        out_shape=jax.ShapeDtypeStruct((npad, 32), jnp.float32),
    )(xp, W10, b1)


def _tc1b_body(h1_ref, d0_ref, d1_ref, ga_ref, gb_ref, dis_ref):
    deg = d0_ref[...] + d1_ref[...]
    dis = jnp.where(deg > 0, lax.rsqrt(deg), 0.0)
    g = dis * h1_ref[...]
    ga_ref[...] = g[:, :16]
    gb_ref[...] = g[:, 16:]
    dis_ref[...] = dis


def _tc1b(h1, d0, d1, npad):
    grid = npad // _BLK
    return pl.pallas_call(
        _tc1b_body,
        grid=(grid,),
        in_specs=[
            pl.BlockSpec((_BLK, 32), lambda i: (i, 0)),
            pl.BlockSpec((_BLK, 1), lambda i: (i, 0)),
            pl.BlockSpec((_BLK, 1), lambda i: (i, 0)),
        ],
        out_specs=[
            pl.BlockSpec((_BLK, 16), lambda i: (i, 0)),
            pl.BlockSpec((_BLK, 16), lambda i: (i, 0)),
            pl.BlockSpec((_BLK, 1), lambda i: (i, 0)),
        ],
        out_shape=[
            jax.ShapeDtypeStruct((npad, 16), jnp.float32),
            jax.ShapeDtypeStruct((npad, 16), jnp.float32),
            jax.ShapeDtypeStruct((npad, 1), jnp.float32),
        ],
    )(h1, d0, d1)


def _tc2_body(h1_ref, qa_ref, qb_ref, dis_ref,
              w20_ref, w21_ref, b2_ref, w30_ref, w31_ref, b3_ref,
              h3a_ref, g20_ref, g21_ref):
    dis = dis_ref[...]
    q1 = jnp.concatenate([qa_ref[...], qb_ref[...]], axis=1)
    p1 = -dis * q1
    h2 = _lrelu(
        jnp.dot(h1_ref[...], w20_ref[...],
                preferred_element_type=jnp.float32)
        + jnp.dot(p1, w21_ref[...], preferred_element_type=jnp.float32)
        + b2_ref[...])
    t = jnp.dot(h2, w31_ref[...], preferred_element_type=jnp.float32)
    g2 = dis * t
    g20_ref[...] = g2[:, 0:1]
    g21_ref[...] = g2[:, 1:2]
    h3a_ref[...] = jnp.dot(h2, w30_ref[...],
                           preferred_element_type=jnp.float32) + b3_ref[...]


def _tc2(h1, qa, qb, dis, W20, W21, b2, W30, W31, b3, npad):
    grid = npad // _BLK
    return pl.pallas_call(
        _tc2_body,
        grid=(grid,),
        in_specs=[
            pl.BlockSpec((_BLK, 32), lambda i: (i, 0)),
            pl.BlockSpec((_BLK, 16), lambda i: (i, 0)),
            pl.BlockSpec((_BLK, 16), lambda i: (i, 0)),
            pl.BlockSpec((_BLK, 1), lambda i: (i, 0)),
            pl.BlockSpec((32, 64), lambda i: (0, 0)),
            pl.BlockSpec((32, 64), lambda i: (0, 0)),
            pl.BlockSpec((1, 64), lambda i: (0, 0)),
            pl.BlockSpec((64, 2), lambda i: (0, 0)),
            pl.BlockSpec((64, 2), lambda i: (0, 0)),
            pl.BlockSpec((1, 2), lambda i: (0, 0)),
        ],
        out_specs=[
            pl.BlockSpec((_BLK, 2), lambda i: (i, 0)),
            pl.BlockSpec((_BLK, 1), lambda i: (i, 0)),
            pl.BlockSpec((_BLK, 1), lambda i: (i, 0)),
        ],
        out_shape=[
            jax.ShapeDtypeStruct((npad, 2), jnp.float32),
            jax.ShapeDtypeStruct((npad, 1), jnp.float32),
            jax.ShapeDtypeStruct((npad, 1), jnp.float32),
        ],
    )(h1, qa, qb, dis, W20, W21, b2, W30, W31, b3)


def _tc3_body(nreal, h3a_ref, o00_ref, o01_ref, o10_ref, o11_ref,
              dis_ref, wg_ref, bg_ref, out_ref, acc_ref):
    i = pl.program_id(0)
    ng = pl.num_programs(0)

    @pl.when(i == 0)
    def _():
        acc_ref[0] = -1e30
        acc_ref[1] = 0.0
        acc_ref[2] = 0.0
        acc_ref[3] = 0.0

    q20 = o00_ref[...] + o01_ref[...]
    q21 = o10_ref[...] + o11_ref[...]
    dis = dis_ref[...]
    h3a = h3a_ref[...]
    h30 = h3a[:, 0:1] - dis * q20
    h31 = h3a[:, 1:2] - dis * q21
    wg = wg_ref[...]
    l = h30 * wg[0, 0] + h31 * wg[0, 1] + bg_ref[0, 0]
    rowid = lax.broadcasted_iota(jnp.int32, l.shape, 0) + i * _BLK
    lm = jnp.where(rowid < nreal, l, -1e30)
    m_old = acc_ref[0]
    m_new = jnp.maximum(m_old, jnp.max(lm))
    e = jnp.exp(lm - m_new)
    scale = jnp.exp(m_old - m_new)
    s_new = acc_ref[1] * scale + jnp.sum(e)
    v0 = acc_ref[2] * scale + jnp.sum(e * h30)
    v1 = acc_ref[3] * scale + jnp.sum(e * h31)
    acc_ref[0] = m_new
    acc_ref[1] = s_new
    acc_ref[2] = v0
    acc_ref[3] = v1

    @pl.when(i == ng - 1)
    def _():
        z0 = acc_ref[2] / acc_ref[1]
        z1 = acc_ref[3] / acc_ref[1]
        mz = jnp.maximum(z0, z1)
        lse = mz + jnp.log(jnp.exp(z0 - mz) + jnp.exp(z1 - mz))
        out_ref[...] = jnp.stack([z0 - lse, z1 - lse]).reshape(1, 2)


def _tc3(h3a, o00, o01, o10, o11, dis, Wg, bg, npad, nreal):
    grid = npad // _BLK
    return pl.pallas_call(
        functools.partial(_tc3_body, nreal),
        grid=(grid,),
        in_specs=[
            pl.BlockSpec((_BLK, 2), lambda i: (i, 0)),
            pl.BlockSpec((_BLK, 1), lambda i: (i, 0)),
            pl.BlockSpec((_BLK, 1), lambda i: (i, 0)),
            pl.BlockSpec((_BLK, 1), lambda i: (i, 0)),
            pl.BlockSpec((_BLK, 1), lambda i: (i, 0)),
            pl.BlockSpec((_BLK, 1), lambda i: (i, 0)),
            pl.BlockSpec((1, 2), lambda i: (0, 0)),
            pl.BlockSpec((1, 1), lambda i: (0, 0)),
        ],
        out_specs=pl.BlockSpec((1, 2), lambda i: (0, 0)),
        out_shape=jax.ShapeDtypeStruct((1, 2), jnp.float32),
        scratch_shapes=[pltpu.SMEM((4,), jnp.float32)],
    )(h3a, o00, o01, o10, o11, dis, Wg, bg)


def _prep_body(e, rows_out, src_r, dst_r, attr_r, src_o, dst_o, attr_o):
    i = pl.program_id(0)
    pos = (i * _PR * 128
           + lax.broadcasted_iota(jnp.int32, (_PR, 128), 0) * 128
           + lax.broadcasted_iota(jnp.int32, (_PR, 128), 1))
    valid = pos < e
    src_o[...] = jnp.where(valid, src_r[...], 0)
    dst_o[...] = jnp.where(valid, dst_r[...], 0)
    attr_o[...] = jnp.where(valid, attr_r[...], 0.0)


_PR = 200


def _prep(src_r, dst_r, attr_r, rows_in, rows_out, e):
    grid = rows_out // _PR
    lim = rows_in // _PR - 1
    spec_in = pl.BlockSpec((_PR, 128), lambda i: (jnp.minimum(i, lim), 0))
    spec_out = pl.BlockSpec((_PR, 128), lambda i: (i, 0))
    return pl.pallas_call(
        functools.partial(_prep_body, e, rows_out),
        grid=(grid,),
        in_specs=[spec_in] * 3,
        out_specs=[spec_out] * 3,
        out_shape=[
            jax.ShapeDtypeStruct((rows_out, 128), jnp.int32),
            jax.ShapeDtypeStruct((rows_out, 128), jnp.int32),
            jax.ShapeDtypeStruct((rows_out, 128), jnp.float32),
        ],
    )(src_r, dst_r, attr_r)


def kernel(x, edge_index, attr, W1, b1, W2, b2, W3, b3, Wg, bg):
    n = x.shape[0]
    e = edge_index.shape[1]
    npad = -(-n // 2048) * 2048
    rows128 = -(-e // 128)
    rows = -(-rows128 // 1024) * 1024
    epad = rows * 128

    if e % (128 * _PR) == 0:
        src2d, dst2d, attr2d = _prep(
            edge_index[0].reshape(rows128, 128),
            edge_index[1].reshape(rows128, 128),
            attr.reshape(rows128, 128), rows128, rows, e)
    else:
        src2d = jnp.pad(edge_index[0], (0, epad - e)).reshape(rows, 128)
        dst2d = jnp.pad(edge_index[1], (0, epad - e)).reshape(rows, 128)
        attr2d = jnp.pad(attr, (0, epad - e)).reshape(rows, 128)

    h1 = _tc1a(x, W1[0], b1.reshape(1, 32), npad)
    d0, d1 = _make_sc_degree(rows, npad, 8)(src2d, attr2d)
    ga, gb, dis = _tc1b(h1, d0.reshape(npad, 1), d1.reshape(npad, 1), npad)
    qa, qb = _make_sc_q1(rows, npad, 2)(src2d, dst2d, attr2d, ga, gb)
    h3a, g20, g21 = _tc2(h1, qa, qb, dis, W2[0], W2[1], b2.reshape(1, 64),
                         W3[0], W3[1], b3.reshape(1, 2), npad)
    o00, o01, o10, o11 = _make_sc_q2(rows, npad, 4)(
        src2d, dst2d, attr2d, g20.reshape(npad), g21.reshape(npad))
    out = _tc3(h3a, o00.reshape(npad, 1), o01.reshape(npad, 1),
               o10.reshape(npad, 1), o11.reshape(npad, 1),
               dis, Wg.reshape(1, 2), bg.reshape(1, 1), npad, n)
    return out
